# Initial kernel scaffold; baseline (speedup 1.0000x reference)
import jax
import jax.numpy as jnp
from jax.experimental import pallas as pl


def kernel(positions, shifts, node_attrs, edge_index, batch, params):
    raise NotImplementedError("write your pallas kernel here")


# speedup vs baseline: 12.5111x; 12.5111x over previous
import functools

import jax
import jax.numpy as jnp
from jax import lax
from jax.experimental import pallas as pl
from jax.experimental.pallas import tpu as pltpu
from jax.experimental.pallas import tpu_sc as plsc

N = 50000
E = 800000
EP = 819200
NG = 32
NB = 8
CUT = 5.0

BN = 2000
BN5 = 1000
BE = 8192
D_TAB = 16
D_MSG = 40
CH = 1024
KSUB = 8
ROWS_W = 200


def _silu(x):
    return x * jax.nn.sigmoid(x)


def _ln(s, g, b):
    mu = jnp.mean(s, axis=-1, keepdims=True)
    var = jnp.mean((s - mu) ** 2, axis=-1, keepdims=True)
    return (s - mu) / jnp.sqrt(var + 1e-5) * g + b


def _full(shape):
    return pl.BlockSpec(shape, lambda i: tuple(0 for _ in shape))


def _node_table_kernel(na_ref, pos_ref, g_ref, b_ref, w_ref, wb_ref, o_ref):
    sn = _ln(na_ref[...], g_ref[...], b_ref[...])
    hs = jnp.dot(sn, w_ref[...], preferred_element_type=jnp.float32) + wb_ref[...]
    o_ref[...] = jnp.concatenate(
        [hs, pos_ref[...], jnp.zeros((hs.shape[0], 5), jnp.float32)], axis=1)


def _node_table(node_attrs, positions, p):
    gv = p['gvp_v']
    return pl.pallas_call(
        _node_table_kernel,
        grid=(N // BN,),
        in_specs=[
            pl.BlockSpec((BN, 4), lambda i: (i, 0)),
            pl.BlockSpec((BN, 3), lambda i: (i, 0)),
            _full((1, 4)), _full((1, 4)), _full((4, 8)), _full((1, 8)),
        ],
        out_specs=pl.BlockSpec((BN, D_TAB), lambda i: (i, 0)),
        out_shape=jax.ShapeDtypeStruct((N, D_TAB), jnp.float32),
    )(node_attrs, positions,
      p['ln_v_g'].reshape(1, 4), p['ln_v_b'].reshape(1, 4),
      gv['ws_w'], gv['ws_b'].reshape(1, 8))


def _sc_gather(table, src2d, dst2d):
    mesh = plsc.VectorSubcoreMesh(core_axis_name="c", subcore_axis_name="s")

    @functools.partial(
        pl.kernel,
        mesh=mesh,
        out_type=(jax.ShapeDtypeStruct((EP, D_TAB), jnp.float32),
                  jax.ShapeDtypeStruct((EP, D_TAB), jnp.float32)),
        scratch_types=[
            pltpu.VMEM((KSUB, 128), jnp.int32),
            pltpu.VMEM((CH, D_TAB), jnp.float32),
            pltpu.SemaphoreType.DMA,
        ],
        compiler_params=pltpu.CompilerParams(use_tc_tiling_on_sc=False),
    )
    def k(tab_hbm, src_hbm, dst_hbm, osrc_hbm, odst_hbm, idx_v, rows_v, sem):
        w = lax.axis_index("c") * 16 + lax.axis_index("s")
        for i_hbm, o_hbm in ((src_hbm, osrc_hbm), (dst_hbm, odst_hbm)):
            @pl.loop(0, ROWS_W // KSUB)
            def _(t):
                r0 = w * ROWS_W + t * KSUB
                pltpu.sync_copy(i_hbm.at[pl.ds(r0, KSUB)], idx_v)
                cps = [
                    pltpu.async_copy(tab_hbm.at[idx_v.at[j]],
                                     rows_v.at[pl.ds(j * 128, 128)], sem)
                    for j in range(KSUB)
                ]
                for cp in cps:
                    cp.wait()
                pltpu.sync_copy(rows_v, o_hbm.at[pl.ds(r0 * 128, CH)])

    return k(table, src2d, dst2d)


def _edge_msg_kernel(sf_ref, df_ref,
                     lneg_ref, lneb_ref, gew_ref, gewb_ref, gegw_ref,
                     scal_ref, wh8sq_ref, m1w_ref, m1wb_ref, m1gw_ref,
                     m1gb_ref, c1_ref, m2h_ref, m2w_ref, m2wb_ref,
                     m2gw_ref, m2gb_ref, m2v_ref, o_ref):
    i = pl.program_id(0)
    sf = sf_ref[...]
    df = df_ref[...]
    a_e = scal_ref[0, 0]
    b_e = scal_ref[0, 1]
    ge_gate_b = scal_ref[0, 2]

    hss, pos_s = sf[:, 0:8], sf[:, 8:11]
    hsd, pos_d = df[:, 0:8], df[:, 8:11]
    vec = pos_d - pos_s
    lsq = jnp.maximum(jnp.sum(vec * vec, axis=1, keepdims=True), 1e-12)
    length = jnp.sqrt(lsq)
    inv_len = 1.0 / length
    unit = vec * inv_len
    wfreq = ((lax.broadcasted_iota(jnp.int32, (1, NB), 1) + 1).astype(jnp.float32)
             * (jnp.pi / CUT))
    bess = jnp.sqrt(2.0 / CUT) * jnp.sin(length * wfreq) * inv_len
    u = length * (1.0 / CUT)
    u3 = u * u * u
    u6 = u3 * u3
    u7 = u6 * u
    u8 = u7 * u
    env = (1.0 - 28.0 * u6 + 48.0 * u7 - 21.0 * u8) * (length < CUT).astype(jnp.float32)
    edge_s = bess * env

    es0 = _ln(edge_s, lneg_ref[...], lneb_ref[...])
    nsq = jnp.sum(unit * unit, axis=1, keepdims=True)
    ev0 = unit / jnp.sqrt(jnp.maximum(nsq, 1e-8))
    ev0sq = jnp.sum(ev0 * ev0, axis=1, keepdims=True)
    vn_e = jnp.sqrt(jnp.maximum(ev0sq * a_e, 1e-8))
    es = jnp.dot(jnp.concatenate([es0, vn_e], axis=1), gew_ref[...],
                 preferred_element_type=jnp.float32) + gewb_ref[...]
    gate_e = jax.nn.sigmoid(
        jnp.sum(es * gegw_ref[...], axis=1, keepdims=True) + ge_gate_b)
    evv = ev0 * (b_e * gate_e)
    evvsq = jnp.sum(evv * evv, axis=1, keepdims=True)

    vn1 = jnp.sqrt(jnp.maximum(evvsq * wh8sq_ref[...], 1e-8))
    s1 = jnp.dot(jnp.concatenate([hss, es, hsd, vn1], axis=1), m1w_ref[...],
                 preferred_element_type=jnp.float32) + m1wb_ref[...]
    gate1 = jax.nn.sigmoid(
        jnp.dot(s1, m1gw_ref[...], preferred_element_type=jnp.float32)
        + m1gb_ref[...])
    g1 = c1_ref[...] * gate1
    s1 = _silu(s1)

    h2 = jnp.dot(g1, m2h_ref[...], preferred_element_type=jnp.float32)
    vn2 = jnp.sqrt(jnp.maximum(evvsq * h2 * h2, 1e-8))
    ms = jnp.dot(jnp.concatenate([s1, vn2], axis=1), m2w_ref[...],
                 preferred_element_type=jnp.float32) + m2wb_ref[...]
    gate2 = jax.nn.sigmoid(
        jnp.dot(ms, m2gw_ref[...], preferred_element_type=jnp.float32)
        + m2gb_ref[...])
    g2 = jnp.dot(h2, m2v_ref[...], preferred_element_type=jnp.float32) * gate2

    gid = lax.broadcasted_iota(jnp.int32, (sf.shape[0], 1), 0) + i * BE
    valid = (gid < E).astype(jnp.float32)
    o_ref[...] = jnp.concatenate(
        [ms, g2 * evv[:, 0:1], g2 * evv[:, 1:2], g2 * evv[:, 2:3],
         valid, jnp.zeros((sf.shape[0], 7), jnp.float32)], axis=1) * valid


def _edge_msg(srcf, dstf, p):
    ge, m1, m2 = p['gvp_e'], p['msg1'], p['msg2']
    wh8 = m1['wh'][8, :]
    scal = jnp.stack([ge['wh'][0, 0] ** 2,
                      ge['wh'][0, 0] * ge['wv'][0, 0],
                      ge['wsv_b'][0]]).reshape(1, 3)
    return pl.pallas_call(
        _edge_msg_kernel,
        grid=(EP // BE,),
        in_specs=[
            pl.BlockSpec((BE, D_TAB), lambda i: (i, 0)),
            pl.BlockSpec((BE, D_TAB), lambda i: (i, 0)),
            _full((1, 8)), _full((1, 8)), _full((9, 8)), _full((1, 8)),
            _full((1, 8)), _full((1, 3)), _full((1, 17)), _full((41, 8)),
            _full((1, 8)), _full((8, 8)), _full((1, 8)), _full((1, 8)),
            _full((8, 8)), _full((16, 8)), _full((1, 8)), _full((8, 8)),
            _full((1, 8)), _full((8, 8)),
        ],
        out_specs=pl.BlockSpec((BE, D_MSG), lambda i: (i, 0)),
        out_shape=jax.ShapeDtypeStruct((EP, D_MSG), jnp.float32),
    )(srcf, dstf,
      p['ln_e_g'].reshape(1, 8), p['ln_e_b'].reshape(1, 8),
      ge['ws_w'], ge['ws_b'].reshape(1, 8), ge['wsv_w'].reshape(1, 8),
      scal, (wh8 * wh8).reshape(1, 17),
      m1['ws_w'], m1['ws_b'].reshape(1, 8), m1['wsv_w'],
      m1['wsv_b'].reshape(1, 8), (wh8 @ m1['wv']).reshape(1, 8),
      m2['wh'], m2['ws_w'], m2['ws_b'].reshape(1, 8), m2['wsv_w'],
      m2['wsv_b'].reshape(1, 8), m2['wv'])


NHALF = N // 2
NACC = 26000
ZROWS = 325
SROWS = NACC // 16


def _sc_scatter(msg, dst2d):
    mesh = plsc.VectorSubcoreMesh(core_axis_name="c", subcore_axis_name="s")
    rows_per_sub = (EP // 128) // 16

    @functools.partial(
        pl.kernel,
        mesh=mesh,
        out_type=jax.ShapeDtypeStruct((2, NACC, D_MSG), jnp.float32),
        scratch_types=[
            pltpu.VMEM((KSUB, 128), jnp.int32),
            pltpu.VMEM((CH, D_MSG), jnp.float32),
            pltpu.VMEM((ZROWS, D_MSG), jnp.float32),
            pltpu.VMEM_SHARED((NACC, D_MSG), jnp.float32),
        ],
        compiler_params=pltpu.CompilerParams(use_tc_tiling_on_sc=False),
    )
    def k(msg_hbm, dst_hbm, out_hbm, idx_v, msg_v, zbuf, acc):
        c = lax.axis_index("c")
        s = lax.axis_index("s")
        lo = c * NHALF
        zero16 = jnp.zeros((16,), jnp.float32)

        @pl.loop(0, ZROWS)
        def _(r):
            zbuf[r, pl.ds(0, 16)] = zero16
            zbuf[r, pl.ds(16, 16)] = zero16
            zbuf[r, pl.ds(24, 16)] = zero16

        @pl.loop(0, 5)
        def _(q):
            pltpu.sync_copy(zbuf, acc.at[pl.ds(s * SROWS + q * ZROWS, ZROWS)])

        plsc.subcore_barrier()

        @pl.loop(0, rows_per_sub // KSUB)
        def _(t):
            r0 = s * rows_per_sub + t * KSUB
            pltpu.sync_copy(dst_hbm.at[pl.ds(r0, KSUB)], idx_v)
            pltpu.sync_copy(msg_hbm.at[pl.ds(r0 * 128, CH)], msg_v)

            @pl.loop(0, KSUB)
            def _(r):
                @pl.loop(0, 8)
                def _(kk):
                    v = idx_v[r, pl.ds(kk * 16, 16)] - lo
                    ok = (v >= 0) & (v < NHALF)
                    idx_v[r, pl.ds(kk * 16, 16)] = jnp.where(ok, v, NHALF)

            for j in range(KSUB):
                pltpu.sync_copy(msg_v.at[pl.ds(j * 128, 128)],
                                acc.at[idx_v.at[j]], add=True)

        plsc.subcore_barrier()

        @pl.loop(0, 5)
        def _(q):
            r = s * SROWS + q * ZROWS
            pltpu.sync_copy(acc.at[pl.ds(r, ZROWS)],
                            out_hbm.at[c, pl.ds(r, ZROWS)])

    return k(msg, dst2d)


def _node_out_kernel(tab_ref, pp_ref, batch_ref,
                     ln0g_ref, ln0b_ref,
                     f1h_ref, f1w_ref, f1wb_ref, f1gw_ref, f1gb_ref, f1v_ref,
                     f2h_ref, f2w_ref, f2wb_ref, f2gw_ref, f2gb_ref, f2v_ref,
                     ln1g_ref, ln1b_ref, lnog_ref, lnob_ref,
                     oh_ref, ow_ref, owb_ref, o_ref, acc_ref, *, nsteps):
    i = pl.program_id(0)

    @pl.when(i == 0)
    def _():
        acc_ref[...] = jnp.zeros_like(acc_ref)

    ps = pp_ref[0]
    hs = tab_ref[:, 0:8]
    cnt = jnp.maximum(ps[:, 32:33], 1.0)
    inv = 1.0 / cnt
    agg_s = ps[:, 0:8] * inv
    V = [ps[:, 8:16] * inv, ps[:, 16:24] * inv, ps[:, 24:32] * inv]

    def gvp_ln(sv, Vv, g, b):
        sn = _ln(sv, g, b)
        nsq = jnp.maximum(Vv[0] ** 2 + Vv[1] ** 2 + Vv[2] ** 2, 1e-8)
        den = 1.0 / jnp.sqrt(jnp.mean(nsq, axis=1, keepdims=True))
        return sn, [v * den for v in Vv]

    def gvp(sv, Vv, wh, ws, wsb, wsv, wsvb, wv, act):
        H = [jnp.dot(v, wh, preferred_element_type=jnp.float32) for v in Vv]
        vn = jnp.sqrt(jnp.maximum(H[0] ** 2 + H[1] ** 2 + H[2] ** 2, 1e-8))
        so = jnp.dot(jnp.concatenate([sv, vn], axis=1), ws,
                     preferred_element_type=jnp.float32) + wsb
        gate = jax.nn.sigmoid(
            jnp.dot(so, wsv, preferred_element_type=jnp.float32) + wsvb)
        Vo = [jnp.dot(h, wv, preferred_element_type=jnp.float32) * gate
              for h in H]
        if act is not None:
            so = act(so)
        return so, Vo

    xs, Xv = gvp_ln(hs + agg_s, V, ln0g_ref[...], ln0b_ref[...])
    fs1, Fv1 = gvp(xs, Xv, f1h_ref[...], f1w_ref[...], f1wb_ref[...],
                   f1gw_ref[...], f1gb_ref[...], f1v_ref[...], _silu)
    fs2, Fv2 = gvp(fs1, Fv1, f2h_ref[...], f2w_ref[...], f2wb_ref[...],
                   f2gw_ref[...], f2gb_ref[...], f2v_ref[...], None)
    ys, Yv = gvp_ln(xs + fs2, [Xv[d] + Fv2[d] for d in range(3)],
                    ln1g_ref[...], ln1b_ref[...])
    os_, Ov = gvp_ln(ys, Yv, lnog_ref[...], lnob_ref[...])
    OH = [jnp.dot(v, oh_ref[...], preferred_element_type=jnp.float32)
          for v in Ov]
    vn3 = jnp.sqrt(jnp.maximum(OH[0] ** 2 + OH[1] ** 2 + OH[2] ** 2, 1e-8))
    out_s = jnp.dot(jnp.concatenate([os_, vn3], axis=1), ow_ref[...],
                    preferred_element_type=jnp.float32) + owb_ref[...]

    bb = batch_ref[0]
    onehot = (lax.broadcasted_iota(jnp.int32, (NG, bb.shape[1]), 0)
              == bb).astype(jnp.float32)
    acc_ref[:, 0:2] += jnp.dot(onehot, out_s, preferred_element_type=jnp.float32)
    acc_ref[:, 2:3] += jnp.sum(onehot, axis=1, keepdims=True)

    @pl.when(i == nsteps - 1)
    def _():
        o_ref[...] = acc_ref[:, 0:2] / jnp.maximum(acc_ref[:, 2:3], 1.0)


def _node_out(table, partials, batch3d, p):
    f1, f2, go = p['ff1'], p['ff2'], p['gvp_out']
    nsteps = N // BN5
    blocks_per_core = NHALF // BN5
    return pl.pallas_call(
        functools.partial(_node_out_kernel, nsteps=nsteps),
        grid=(nsteps,),
        in_specs=[
            pl.BlockSpec((BN5, D_TAB), lambda i: (i, 0)),
            pl.BlockSpec((1, BN5, D_MSG),
                         lambda i: (i // blocks_per_core,
                                    i % blocks_per_core, 0)),
            pl.BlockSpec((1, 1, BN5), lambda i: (i, 0, 0)),
            _full((1, 8)), _full((1, 8)),
            _full((8, 16)), _full((24, 32)), _full((1, 32)), _full((32, 16)),
            _full((1, 16)), _full((16, 16)),
            _full((16, 16)), _full((48, 8)), _full((1, 8)), _full((8, 8)),
            _full((1, 8)), _full((16, 8)),
            _full((1, 8)), _full((1, 8)), _full((1, 8)), _full((1, 8)),
            _full((8, 8)), _full((16, 2)), _full((1, 2)),
        ],
        out_specs=pl.BlockSpec((NG, 2), lambda i: (0, 0)),
        out_shape=jax.ShapeDtypeStruct((NG, 2), jnp.float32),
        scratch_shapes=[pltpu.VMEM((NG, 128), jnp.float32)],
    )(table, partials, batch3d,
      p['ln0_g'].reshape(1, 8), p['ln0_b'].reshape(1, 8),
      f1['wh'], f1['ws_w'], f1['ws_b'].reshape(1, 32), f1['wsv_w'],
      f1['wsv_b'].reshape(1, 16), f1['wv'],
      f2['wh'], f2['ws_w'], f2['ws_b'].reshape(1, 8), f2['wsv_w'],
      f2['wsv_b'].reshape(1, 8), f2['wv'],
      p['ln1_g'].reshape(1, 8), p['ln1_b'].reshape(1, 8),
      p['ln_out_g'].reshape(1, 8), p['ln_out_b'].reshape(1, 8),
      go['wh'], go['ws_w'], go['ws_b'].reshape(1, 2))


def kernel(positions, shifts, node_attrs, edge_index, batch, params):
    src = edge_index[0]
    dst = edge_index[1]
    pad = jnp.zeros((EP - E,), jnp.int32)
    src2d = jnp.concatenate([src, pad]).reshape(EP // 128, 128)
    dst2d = jnp.concatenate([dst, pad]).reshape(EP // 128, 128)

    table = _node_table(node_attrs, positions, params)
    srcf, dstf = _sc_gather(table, src2d, dst2d)
    msg = _edge_msg(srcf, dstf, params)
    partials = _sc_scatter(msg, dst2d)
    return _node_out(table, partials, batch.reshape(N // BN5, 1, BN5), params)


# speedup vs baseline: 31.8327x; 2.5443x over previous
import functools

import jax
import jax.numpy as jnp
from jax import lax
from jax.experimental import pallas as pl
from jax.experimental.pallas import tpu as pltpu
from jax.experimental.pallas import tpu_sc as plsc

N = 50000
E = 800000
EP = 819200
NG = 32
NB = 8
CUT = 5.0

BN = 2000
BN5 = 1000
BE = 8192
D_TAB = 16
D_MSG = 40
CH = 1024
KSUB = 8
ROWS_W = 200


def _silu(x):
    return x * jax.nn.sigmoid(x)


def _ln(s, g, b):
    mu = jnp.mean(s, axis=-1, keepdims=True)
    var = jnp.mean((s - mu) ** 2, axis=-1, keepdims=True)
    return (s - mu) / jnp.sqrt(var + 1e-5) * g + b


def _full(shape):
    return pl.BlockSpec(shape, lambda i: tuple(0 for _ in shape))


def _node_table_kernel(na_ref, pos_ref, g_ref, b_ref, w_ref, wb_ref, o_ref):
    sn = _ln(na_ref[...], g_ref[...], b_ref[...])
    hs = jnp.dot(sn, w_ref[...], preferred_element_type=jnp.float32) + wb_ref[...]
    o_ref[...] = jnp.concatenate(
        [hs, pos_ref[...], jnp.zeros((hs.shape[0], 5), jnp.float32)], axis=1)


def _node_table(node_attrs, positions, p):
    gv = p['gvp_v']
    return pl.pallas_call(
        _node_table_kernel,
        grid=(N // BN,),
        in_specs=[
            pl.BlockSpec((BN, 4), lambda i: (i, 0)),
            pl.BlockSpec((BN, 3), lambda i: (i, 0)),
            _full((1, 4)), _full((1, 4)), _full((4, 8)), _full((1, 8)),
        ],
        out_specs=pl.BlockSpec((BN, D_TAB), lambda i: (i, 0)),
        out_shape=jax.ShapeDtypeStruct((N, D_TAB), jnp.float32),
    )(node_attrs, positions,
      p['ln_v_g'].reshape(1, 4), p['ln_v_b'].reshape(1, 4),
      gv['ws_w'], gv['ws_b'].reshape(1, 8))


def _sc_gather(table, src2d, dst2d):
    mesh = plsc.VectorSubcoreMesh(core_axis_name="c", subcore_axis_name="s")

    @functools.partial(
        pl.kernel,
        mesh=mesh,
        out_type=(jax.ShapeDtypeStruct((EP, D_TAB), jnp.float32),
                  jax.ShapeDtypeStruct((EP, D_TAB), jnp.float32)),
        scratch_types=[
            pltpu.VMEM((KSUB, 128), jnp.int32),
            pltpu.VMEM((CH, D_TAB), jnp.float32),
            pltpu.SemaphoreType.DMA,
        ],
        compiler_params=pltpu.CompilerParams(use_tc_tiling_on_sc=False),
    )
    def k(tab_hbm, src_hbm, dst_hbm, osrc_hbm, odst_hbm, idx_v, rows_v, sem):
        w = lax.axis_index("c") * 16 + lax.axis_index("s")
        for i_hbm, o_hbm in ((src_hbm, osrc_hbm), (dst_hbm, odst_hbm)):
            @pl.loop(0, ROWS_W // KSUB)
            def _(t):
                r0 = w * ROWS_W + t * KSUB
                pltpu.sync_copy(i_hbm.at[pl.ds(r0, KSUB)], idx_v)
                cps = [
                    pltpu.async_copy(tab_hbm.at[idx_v.at[j]],
                                     rows_v.at[pl.ds(j * 128, 128)], sem)
                    for j in range(KSUB)
                ]
                for cp in cps:
                    cp.wait()
                pltpu.sync_copy(rows_v, o_hbm.at[pl.ds(r0 * 128, CH)])

    return k(table, src2d, dst2d)


def _dotT(w, x):
    return lax.dot_general(w, x, (((0,), (0,)), ((), ())),
                           preferred_element_type=jnp.float32)


def _edge_msg_kernel(sf_ref, df_ref,
                     lneg_ref, lneb_ref, gew_ref, gewb_ref, gegw_ref,
                     scal_ref, wh8sq_ref, m1w_ref, m1wb_ref, m1gw_ref,
                     m1gb_ref, c1_ref, m2h_ref, m2w_ref, m2wb_ref,
                     m2gw_ref, m2gb_ref, m2v_ref, o_ref):
    i = pl.program_id(0)
    sfT = sf_ref[...].T
    dfT = df_ref[...].T
    a_e = scal_ref[0, 0]
    b_e = scal_ref[0, 1]
    ge_gate_b = scal_ref[0, 2]

    hss, pos_s = sfT[0:8], sfT[8:11]
    hsd, pos_d = dfT[0:8], dfT[8:11]
    vec = pos_d - pos_s
    lsq = jnp.maximum(jnp.sum(vec * vec, axis=0, keepdims=True), 1e-12)
    length = jnp.sqrt(lsq)
    inv_len = 1.0 / length
    unit = vec * inv_len
    wfreq = ((lax.broadcasted_iota(jnp.int32, (NB, 1), 0) + 1).astype(jnp.float32)
             * (jnp.pi / CUT))
    bess = jnp.sqrt(2.0 / CUT) * jnp.sin(length * wfreq) * inv_len
    u = length * (1.0 / CUT)
    u3 = u * u * u
    u6 = u3 * u3
    u7 = u6 * u
    u8 = u7 * u
    env = (1.0 - 28.0 * u6 + 48.0 * u7 - 21.0 * u8) * (length < CUT).astype(jnp.float32)
    edge_s = bess * env

    mu = jnp.mean(edge_s, axis=0, keepdims=True)
    var = jnp.mean((edge_s - mu) ** 2, axis=0, keepdims=True)
    es0 = (edge_s - mu) / jnp.sqrt(var + 1e-5) * lneg_ref[...] + lneb_ref[...]
    nsq = jnp.sum(unit * unit, axis=0, keepdims=True)
    ev0 = unit / jnp.sqrt(jnp.maximum(nsq, 1e-8))
    ev0sq = jnp.sum(ev0 * ev0, axis=0, keepdims=True)
    vn_e = jnp.sqrt(jnp.maximum(ev0sq * a_e, 1e-8))
    es = _dotT(gew_ref[...], jnp.concatenate([es0, vn_e], axis=0)) \
        + gewb_ref[...]
    gate_e = jax.nn.sigmoid(
        jnp.sum(es * gegw_ref[...], axis=0, keepdims=True) + ge_gate_b)
    evv = ev0 * (b_e * gate_e)
    evvsq = jnp.sum(evv * evv, axis=0, keepdims=True)

    vn1 = jnp.sqrt(jnp.maximum(evvsq * wh8sq_ref[...], 1e-8))
    x41 = jnp.concatenate([hss, es, hsd, vn1], axis=0)
    s1 = _dotT(m1w_ref[...], x41) + m1wb_ref[...]
    gate1 = jax.nn.sigmoid(_dotT(m1gw_ref[...], s1) + m1gb_ref[...])
    g1 = c1_ref[...] * gate1
    s1 = _silu(s1)

    h2 = _dotT(m2h_ref[...], g1)
    vn2 = jnp.sqrt(jnp.maximum(evvsq * h2 * h2, 1e-8))
    ms = _dotT(m2w_ref[...], jnp.concatenate([s1, vn2], axis=0)) \
        + m2wb_ref[...]
    gate2 = jax.nn.sigmoid(_dotT(m2gw_ref[...], ms) + m2gb_ref[...])
    g2 = _dotT(m2v_ref[...], h2) * gate2

    gid = lax.broadcasted_iota(jnp.int32, (1, sfT.shape[1]), 1) + i * BE
    valid = (gid < E).astype(jnp.float32)
    outT = jnp.concatenate(
        [ms, g2 * evv[0:1], g2 * evv[1:2], g2 * evv[2:3],
         valid, jnp.zeros((7, sfT.shape[1]), jnp.float32)], axis=0) * valid
    o_ref[...] = outT.T


def _edge_msg(srcf, dstf, p):
    ge, m1, m2 = p['gvp_e'], p['msg1'], p['msg2']
    wh8 = m1['wh'][8, :]
    scal = jnp.stack([ge['wh'][0, 0] ** 2,
                      ge['wh'][0, 0] * ge['wv'][0, 0],
                      ge['wsv_b'][0]]).reshape(1, 3)
    return pl.pallas_call(
        _edge_msg_kernel,
        grid=(EP // BE,),
        in_specs=[
            pl.BlockSpec((BE, D_TAB), lambda i: (i, 0)),
            pl.BlockSpec((BE, D_TAB), lambda i: (i, 0)),
            _full((8, 1)), _full((8, 1)), _full((9, 8)), _full((8, 1)),
            _full((8, 1)), _full((1, 3)), _full((17, 1)), _full((41, 8)),
            _full((8, 1)), _full((8, 8)), _full((8, 1)), _full((8, 1)),
            _full((8, 8)), _full((16, 8)), _full((8, 1)), _full((8, 8)),
            _full((8, 1)), _full((8, 8)),
        ],
        out_specs=pl.BlockSpec((BE, D_MSG), lambda i: (i, 0)),
        out_shape=jax.ShapeDtypeStruct((EP, D_MSG), jnp.float32),
    )(srcf, dstf,
      p['ln_e_g'].reshape(8, 1), p['ln_e_b'].reshape(8, 1),
      ge['ws_w'], ge['ws_b'].reshape(8, 1), ge['wsv_w'].reshape(8, 1),
      scal, (wh8 * wh8).reshape(17, 1),
      m1['ws_w'], m1['ws_b'].reshape(8, 1), m1['wsv_w'],
      m1['wsv_b'].reshape(8, 1), (wh8 @ m1['wv']).reshape(8, 1),
      m2['wh'], m2['ws_w'], m2['ws_b'].reshape(8, 1), m2['wsv_w'],
      m2['wsv_b'].reshape(8, 1), m2['wv'])


NHALF = N // 2
NACC = 26000
ZROWS = 325
SROWS = NACC // 16


def _sc_scatter(msg, dst2d):
    mesh = plsc.VectorSubcoreMesh(core_axis_name="c", subcore_axis_name="s")
    rows_per_sub = (EP // 128) // 16

    @functools.partial(
        pl.kernel,
        mesh=mesh,
        out_type=jax.ShapeDtypeStruct((2, NACC, D_MSG), jnp.float32),
        scratch_types=[
            pltpu.VMEM((KSUB, 128), jnp.int32),
            pltpu.VMEM((CH, D_MSG), jnp.float32),
            pltpu.VMEM((ZROWS, D_MSG), jnp.float32),
            pltpu.VMEM_SHARED((NACC, D_MSG), jnp.float32),
        ],
        compiler_params=pltpu.CompilerParams(use_tc_tiling_on_sc=False),
    )
    def k(msg_hbm, dst_hbm, out_hbm, idx_v, msg_v, zbuf, acc):
        c = lax.axis_index("c")
        s = lax.axis_index("s")
        lo = c * NHALF
        zero16 = jnp.zeros((16,), jnp.float32)

        @pl.loop(0, ZROWS)
        def _(r):
            zbuf[r, pl.ds(0, 16)] = zero16
            zbuf[r, pl.ds(16, 16)] = zero16
            zbuf[r, pl.ds(24, 16)] = zero16

        @pl.loop(0, 5)
        def _(q):
            pltpu.sync_copy(zbuf, acc.at[pl.ds(s * SROWS + q * ZROWS, ZROWS)])

        plsc.subcore_barrier()

        @pl.loop(0, rows_per_sub // KSUB)
        def _(t):
            r0 = s * rows_per_sub + t * KSUB
            pltpu.sync_copy(dst_hbm.at[pl.ds(r0, KSUB)], idx_v)
            pltpu.sync_copy(msg_hbm.at[pl.ds(r0 * 128, CH)], msg_v)

            @pl.loop(0, KSUB)
            def _(r):
                @pl.loop(0, 8)
                def _(kk):
                    v = idx_v[r, pl.ds(kk * 16, 16)] - lo
                    ok = (v >= 0) & (v < NHALF)
                    idx_v[r, pl.ds(kk * 16, 16)] = jnp.where(ok, v, NHALF)

            for j in range(KSUB):
                pltpu.sync_copy(msg_v.at[pl.ds(j * 128, 128)],
                                acc.at[idx_v.at[j]], add=True)

        plsc.subcore_barrier()

        @pl.loop(0, 5)
        def _(q):
            r = s * SROWS + q * ZROWS
            pltpu.sync_copy(acc.at[pl.ds(r, ZROWS)],
                            out_hbm.at[c, pl.ds(r, ZROWS)])

    return k(msg, dst2d)


def _node_out_kernel(tab_ref, pp_ref, batch_ref,
                     ln0g_ref, ln0b_ref,
                     f1h_ref, f1w_ref, f1wb_ref, f1gw_ref, f1gb_ref, f1v_ref,
                     f2h_ref, f2w_ref, f2wb_ref, f2gw_ref, f2gb_ref, f2v_ref,
                     ln1g_ref, ln1b_ref, lnog_ref, lnob_ref,
                     oh_ref, ow_ref, owb_ref, o_ref, acc_ref, *, nsteps):
    i = pl.program_id(0)

    @pl.when(i == 0)
    def _():
        acc_ref[...] = jnp.zeros_like(acc_ref)

    ps = pp_ref[0]
    hs = tab_ref[:, 0:8]
    cnt = jnp.maximum(ps[:, 32:33], 1.0)
    inv = 1.0 / cnt
    agg_s = ps[:, 0:8] * inv
    V = [ps[:, 8:16] * inv, ps[:, 16:24] * inv, ps[:, 24:32] * inv]

    def gvp_ln(sv, Vv, g, b):
        sn = _ln(sv, g, b)
        nsq = jnp.maximum(Vv[0] ** 2 + Vv[1] ** 2 + Vv[2] ** 2, 1e-8)
        den = 1.0 / jnp.sqrt(jnp.mean(nsq, axis=1, keepdims=True))
        return sn, [v * den for v in Vv]

    def gvp(sv, Vv, wh, ws, wsb, wsv, wsvb, wv, act):
        H = [jnp.dot(v, wh, preferred_element_type=jnp.float32) for v in Vv]
        vn = jnp.sqrt(jnp.maximum(H[0] ** 2 + H[1] ** 2 + H[2] ** 2, 1e-8))
        so = jnp.dot(jnp.concatenate([sv, vn], axis=1), ws,
                     preferred_element_type=jnp.float32) + wsb
        gate = jax.nn.sigmoid(
            jnp.dot(so, wsv, preferred_element_type=jnp.float32) + wsvb)
        Vo = [jnp.dot(h, wv, preferred_element_type=jnp.float32) * gate
              for h in H]
        if act is not None:
            so = act(so)
        return so, Vo

    xs, Xv = gvp_ln(hs + agg_s, V, ln0g_ref[...], ln0b_ref[...])
    fs1, Fv1 = gvp(xs, Xv, f1h_ref[...], f1w_ref[...], f1wb_ref[...],
                   f1gw_ref[...], f1gb_ref[...], f1v_ref[...], _silu)
    fs2, Fv2 = gvp(fs1, Fv1, f2h_ref[...], f2w_ref[...], f2wb_ref[...],
                   f2gw_ref[...], f2gb_ref[...], f2v_ref[...], None)
    ys, Yv = gvp_ln(xs + fs2, [Xv[d] + Fv2[d] for d in range(3)],
                    ln1g_ref[...], ln1b_ref[...])
    os_, Ov = gvp_ln(ys, Yv, lnog_ref[...], lnob_ref[...])
    OH = [jnp.dot(v, oh_ref[...], preferred_element_type=jnp.float32)
          for v in Ov]
    vn3 = jnp.sqrt(jnp.maximum(OH[0] ** 2 + OH[1] ** 2 + OH[2] ** 2, 1e-8))
    out_s = jnp.dot(jnp.concatenate([os_, vn3], axis=1), ow_ref[...],
                    preferred_element_type=jnp.float32) + owb_ref[...]

    bb = batch_ref[0]
    onehot = (lax.broadcasted_iota(jnp.int32, (NG, bb.shape[1]), 0)
              == bb).astype(jnp.float32)
    acc_ref[:, 0:2] += jnp.dot(onehot, out_s, preferred_element_type=jnp.float32)
    acc_ref[:, 2:3] += jnp.sum(onehot, axis=1, keepdims=True)

    @pl.when(i == nsteps - 1)
    def _():
        o_ref[...] = acc_ref[:, 0:2] / jnp.maximum(acc_ref[:, 2:3], 1.0)


def _node_out(table, partials, batch3d, p):
    f1, f2, go = p['ff1'], p['ff2'], p['gvp_out']
    nsteps = N // BN5
    blocks_per_core = NHALF // BN5
    return pl.pallas_call(
        functools.partial(_node_out_kernel, nsteps=nsteps),
        grid=(nsteps,),
        in_specs=[
            pl.BlockSpec((BN5, D_TAB), lambda i: (i, 0)),
            pl.BlockSpec((1, BN5, D_MSG),
                         lambda i: (i // blocks_per_core,
                                    i % blocks_per_core, 0)),
            pl.BlockSpec((1, 1, BN5), lambda i: (i, 0, 0)),
            _full((1, 8)), _full((1, 8)),
            _full((8, 16)), _full((24, 32)), _full((1, 32)), _full((32, 16)),
            _full((1, 16)), _full((16, 16)),
            _full((16, 16)), _full((48, 8)), _full((1, 8)), _full((8, 8)),
            _full((1, 8)), _full((16, 8)),
            _full((1, 8)), _full((1, 8)), _full((1, 8)), _full((1, 8)),
            _full((8, 8)), _full((16, 2)), _full((1, 2)),
        ],
        out_specs=pl.BlockSpec((NG, 2), lambda i: (0, 0)),
        out_shape=jax.ShapeDtypeStruct((NG, 2), jnp.float32),
        scratch_shapes=[pltpu.VMEM((NG, 128), jnp.float32)],
    )(table, partials, batch3d,
      p['ln0_g'].reshape(1, 8), p['ln0_b'].reshape(1, 8),
      f1['wh'], f1['ws_w'], f1['ws_b'].reshape(1, 32), f1['wsv_w'],
      f1['wsv_b'].reshape(1, 16), f1['wv'],
      f2['wh'], f2['ws_w'], f2['ws_b'].reshape(1, 8), f2['wsv_w'],
      f2['wsv_b'].reshape(1, 8), f2['wv'],
      p['ln1_g'].reshape(1, 8), p['ln1_b'].reshape(1, 8),
      p['ln_out_g'].reshape(1, 8), p['ln_out_b'].reshape(1, 8),
      go['wh'], go['ws_w'], go['ws_b'].reshape(1, 2))


def kernel(positions, shifts, node_attrs, edge_index, batch, params):
    src = edge_index[0]
    dst = edge_index[1]
    pad = jnp.zeros((EP - E,), jnp.int32)
    src2d = jnp.concatenate([src, pad]).reshape(EP // 128, 128)
    dst2d = jnp.concatenate([dst, pad]).reshape(EP // 128, 128)

    table = _node_table(node_attrs, positions, params)
    srcf, dstf = _sc_gather(table, src2d, dst2d)
    msg = _edge_msg(srcf, dstf, params)
    partials = _sc_scatter(msg, dst2d)
    return _node_out(table, partials, batch.reshape(N // BN5, 1, BN5), params)


# speedup vs baseline: 38.7369x; 1.2169x over previous
import functools

import jax
import jax.numpy as jnp
from jax import lax
from jax.experimental import pallas as pl
from jax.experimental.pallas import tpu as pltpu
from jax.experimental.pallas import tpu_sc as plsc

N = 50000
E = 800000
EP = 819200
NG = 32
NB = 8
CUT = 5.0

BN = 2000
BN5 = 1000
BE = 8192
D_TAB = 16
D_MSG = 40
CH = 1024
KSUB = 8
ROWS_W = 200


def _silu(x):
    return x * jax.nn.sigmoid(x)


def _ln(s, g, b):
    mu = jnp.mean(s, axis=-1, keepdims=True)
    var = jnp.mean((s - mu) ** 2, axis=-1, keepdims=True)
    return (s - mu) / jnp.sqrt(var + 1e-5) * g + b


def _full(shape):
    return pl.BlockSpec(shape, lambda i: tuple(0 for _ in shape))


def _node_table_kernel(na_ref, pos_ref, g_ref, b_ref, w_ref, wb_ref, o_ref):
    sn = _ln(na_ref[...], g_ref[...], b_ref[...])
    hs = jnp.dot(sn, w_ref[...], preferred_element_type=jnp.float32) + wb_ref[...]
    o_ref[...] = jnp.concatenate(
        [hs, pos_ref[...], jnp.zeros((hs.shape[0], 5), jnp.float32)], axis=1)


def _node_table(node_attrs, positions, p):
    gv = p['gvp_v']
    return pl.pallas_call(
        _node_table_kernel,
        grid=(N // BN,),
        in_specs=[
            pl.BlockSpec((BN, 4), lambda i: (i, 0)),
            pl.BlockSpec((BN, 3), lambda i: (i, 0)),
            _full((1, 4)), _full((1, 4)), _full((4, 8)), _full((1, 8)),
        ],
        out_specs=pl.BlockSpec((BN, D_TAB), lambda i: (i, 0)),
        out_shape=jax.ShapeDtypeStruct((N, D_TAB), jnp.float32),
    )(node_attrs, positions,
      p['ln_v_g'].reshape(1, 4), p['ln_v_b'].reshape(1, 4),
      gv['ws_w'], gv['ws_b'].reshape(1, 8))


def _sc_gather(table, src2d, dst2d):
    mesh = plsc.VectorSubcoreMesh(core_axis_name="c", subcore_axis_name="s")

    @functools.partial(
        pl.kernel,
        mesh=mesh,
        out_type=(jax.ShapeDtypeStruct((EP, D_TAB), jnp.float32),
                  jax.ShapeDtypeStruct((EP, D_TAB), jnp.float32)),
        scratch_types=[
            pltpu.VMEM((KSUB, 128), jnp.int32),
            pltpu.VMEM((CH, D_TAB), jnp.float32),
            pltpu.SemaphoreType.DMA,
        ],
        compiler_params=pltpu.CompilerParams(use_tc_tiling_on_sc=False),
    )
    def k(tab_hbm, src_hbm, dst_hbm, osrc_hbm, odst_hbm, idx_v, rows_v, sem):
        w = lax.axis_index("c") * 16 + lax.axis_index("s")
        for i_hbm, o_hbm in ((src_hbm, osrc_hbm), (dst_hbm, odst_hbm)):
            @pl.loop(0, ROWS_W // KSUB)
            def _(t):
                r0 = w * ROWS_W + t * KSUB
                pltpu.sync_copy(i_hbm.at[pl.ds(r0, KSUB)], idx_v)
                cps = [
                    pltpu.async_copy(tab_hbm.at[idx_v.at[j]],
                                     rows_v.at[pl.ds(j * 128, 128)], sem)
                    for j in range(KSUB)
                ]
                for cp in cps:
                    cp.wait()
                pltpu.sync_copy(rows_v, o_hbm.at[pl.ds(r0 * 128, CH)])

    return k(table, src2d, dst2d)


def _dotT(w, x):
    return lax.dot_general(w, x, (((0,), (0,)), ((), ())),
                           preferred_element_type=jnp.float32)


def _edge_msg_kernel(sf_ref, df_ref,
                     lneg_ref, lneb_ref, gew_ref, gewb_ref, gegw_ref,
                     scal_ref, wh8sq_ref, m1w_ref, m1wb_ref, m1gw_ref,
                     m1gb_ref, c1_ref, m2h_ref, m2w_ref, m2wb_ref,
                     m2gw_ref, m2gb_ref, m2v_ref, o_ref):
    i = pl.program_id(0)
    sfT = sf_ref[...].T
    dfT = df_ref[...].T
    a_e = scal_ref[0, 0]
    b_e = scal_ref[0, 1]
    ge_gate_b = scal_ref[0, 2]

    hss, pos_s = sfT[0:8], sfT[8:11]
    hsd, pos_d = dfT[0:8], dfT[8:11]
    vec = pos_d - pos_s
    lsq = jnp.maximum(jnp.sum(vec * vec, axis=0, keepdims=True), 1e-12)
    length = jnp.sqrt(lsq)
    inv_len = 1.0 / length
    unit = vec * inv_len
    wfreq = ((lax.broadcasted_iota(jnp.int32, (NB, 1), 0) + 1).astype(jnp.float32)
             * (jnp.pi / CUT))
    bess = jnp.sqrt(2.0 / CUT) * jnp.sin(length * wfreq) * inv_len
    u = length * (1.0 / CUT)
    u3 = u * u * u
    u6 = u3 * u3
    u7 = u6 * u
    u8 = u7 * u
    env = (1.0 - 28.0 * u6 + 48.0 * u7 - 21.0 * u8) * (length < CUT).astype(jnp.float32)
    edge_s = bess * env

    mu = jnp.mean(edge_s, axis=0, keepdims=True)
    var = jnp.mean((edge_s - mu) ** 2, axis=0, keepdims=True)
    es0 = (edge_s - mu) / jnp.sqrt(var + 1e-5) * lneg_ref[...] + lneb_ref[...]
    nsq = jnp.sum(unit * unit, axis=0, keepdims=True)
    ev0 = unit / jnp.sqrt(jnp.maximum(nsq, 1e-8))
    ev0sq = jnp.sum(ev0 * ev0, axis=0, keepdims=True)
    vn_e = jnp.sqrt(jnp.maximum(ev0sq * a_e, 1e-8))
    es = _dotT(gew_ref[...], jnp.concatenate([es0, vn_e], axis=0)) \
        + gewb_ref[...]
    gate_e = jax.nn.sigmoid(
        jnp.sum(es * gegw_ref[...], axis=0, keepdims=True) + ge_gate_b)
    evv = ev0 * (b_e * gate_e)
    evvsq = jnp.sum(evv * evv, axis=0, keepdims=True)

    vn1 = jnp.sqrt(jnp.maximum(evvsq * wh8sq_ref[...], 1e-8))
    x41 = jnp.concatenate([hss, es, hsd, vn1], axis=0)
    s1 = _dotT(m1w_ref[...], x41) + m1wb_ref[...]
    gate1 = jax.nn.sigmoid(_dotT(m1gw_ref[...], s1) + m1gb_ref[...])
    g1 = c1_ref[...] * gate1
    s1 = _silu(s1)

    h2 = _dotT(m2h_ref[...], g1)
    vn2 = jnp.sqrt(jnp.maximum(evvsq * h2 * h2, 1e-8))
    ms = _dotT(m2w_ref[...], jnp.concatenate([s1, vn2], axis=0)) \
        + m2wb_ref[...]
    gate2 = jax.nn.sigmoid(_dotT(m2gw_ref[...], ms) + m2gb_ref[...])
    g2 = _dotT(m2v_ref[...], h2) * gate2

    gid = lax.broadcasted_iota(jnp.int32, (1, sfT.shape[1]), 1) + i * BE
    valid = (gid < E).astype(jnp.float32)
    outT = jnp.concatenate(
        [ms, g2 * evv[0:1], g2 * evv[1:2], g2 * evv[2:3],
         valid, jnp.zeros((7, sfT.shape[1]), jnp.float32)], axis=0) * valid
    o_ref[...] = outT.T


def _edge_msg(srcf, dstf, p):
    ge, m1, m2 = p['gvp_e'], p['msg1'], p['msg2']
    wh8 = m1['wh'][8, :]
    scal = jnp.stack([ge['wh'][0, 0] ** 2,
                      ge['wh'][0, 0] * ge['wv'][0, 0],
                      ge['wsv_b'][0]]).reshape(1, 3)
    return pl.pallas_call(
        _edge_msg_kernel,
        grid=(EP // BE,),
        in_specs=[
            pl.BlockSpec((BE, D_TAB), lambda i: (i, 0)),
            pl.BlockSpec((BE, D_TAB), lambda i: (i, 0)),
            _full((8, 1)), _full((8, 1)), _full((9, 8)), _full((8, 1)),
            _full((8, 1)), _full((1, 3)), _full((17, 1)), _full((41, 8)),
            _full((8, 1)), _full((8, 8)), _full((8, 1)), _full((8, 1)),
            _full((8, 8)), _full((16, 8)), _full((8, 1)), _full((8, 8)),
            _full((8, 1)), _full((8, 8)),
        ],
        out_specs=pl.BlockSpec((BE, D_MSG), lambda i: (i, 0)),
        out_shape=jax.ShapeDtypeStruct((EP, D_MSG), jnp.float32),
    )(srcf, dstf,
      p['ln_e_g'].reshape(8, 1), p['ln_e_b'].reshape(8, 1),
      ge['ws_w'], ge['ws_b'].reshape(8, 1), ge['wsv_w'].reshape(8, 1),
      scal, (wh8 * wh8).reshape(17, 1),
      m1['ws_w'], m1['ws_b'].reshape(8, 1), m1['wsv_w'],
      m1['wsv_b'].reshape(8, 1), (wh8 @ m1['wv']).reshape(8, 1),
      m2['wh'], m2['ws_w'], m2['ws_b'].reshape(8, 1), m2['wsv_w'],
      m2['wsv_b'].reshape(8, 1), m2['wv'])


NHALF = N // 2
NACC = 26000
ZROWS = 325
SROWS = NACC // 16


def _sc_scatter(msg, dst2d):
    mesh = plsc.VectorSubcoreMesh(core_axis_name="c", subcore_axis_name="s")
    rows_per_sub = (EP // 128) // 16

    @functools.partial(
        pl.kernel,
        mesh=mesh,
        out_type=jax.ShapeDtypeStruct((2, NACC, D_MSG), jnp.float32),
        scratch_types=[
            pltpu.VMEM((KSUB, 128), jnp.int32),
            pltpu.VMEM((CH, D_MSG), jnp.float32),
            pltpu.VMEM((ZROWS, D_MSG), jnp.float32),
            pltpu.VMEM_SHARED((NACC, D_MSG), jnp.float32),
        ],
        compiler_params=pltpu.CompilerParams(use_tc_tiling_on_sc=False),
    )
    def k(msg_hbm, dst_hbm, out_hbm, idx_v, msg_v, zbuf, acc):
        c = lax.axis_index("c")
        s = lax.axis_index("s")
        lo = c * NHALF
        zero16 = jnp.zeros((16,), jnp.float32)

        @pl.loop(0, ZROWS)
        def _(r):
            zbuf[r, pl.ds(0, 16)] = zero16
            zbuf[r, pl.ds(16, 16)] = zero16
            zbuf[r, pl.ds(24, 16)] = zero16

        @pl.loop(0, 5)
        def _(q):
            pltpu.sync_copy(zbuf, acc.at[pl.ds(s * SROWS + q * ZROWS, ZROWS)])

        plsc.subcore_barrier()

        @pl.loop(0, rows_per_sub // KSUB)
        def _(t):
            r0 = s * rows_per_sub + t * KSUB
            pltpu.sync_copy(dst_hbm.at[pl.ds(r0, KSUB)], idx_v)
            pltpu.sync_copy(msg_hbm.at[pl.ds(r0 * 128, CH)], msg_v)

            @pl.loop(0, KSUB)
            def _(r):
                @pl.loop(0, 8)
                def _(kk):
                    v = idx_v[r, pl.ds(kk * 16, 16)] - lo
                    ok = (v >= 0) & (v < NHALF)
                    idx_v[r, pl.ds(kk * 16, 16)] = jnp.where(ok, v, NHALF)

            for j in range(KSUB):
                pltpu.sync_copy(msg_v.at[pl.ds(j * 128, 128)],
                                acc.at[idx_v.at[j]], add=True)

        plsc.subcore_barrier()

        @pl.loop(0, 5)
        def _(q):
            r = s * SROWS + q * ZROWS
            pltpu.sync_copy(acc.at[pl.ds(r, ZROWS)],
                            out_hbm.at[c, pl.ds(r, ZROWS)])

    return k(msg, dst2d)


def _node_out_kernel(tab_ref, pp_ref, batch_ref,
                     ln0g_ref, ln0b_ref,
                     f1h_ref, f1w_ref, f1wb_ref, f1gw_ref, f1gb_ref, f1v_ref,
                     f2h_ref, f2w_ref, f2wb_ref, f2gw_ref, f2gb_ref, f2v_ref,
                     ln1g_ref, ln1b_ref, lnog_ref, lnob_ref,
                     oh_ref, ow_ref, owb_ref, o_ref, acc_ref, *, nsteps):
    i = pl.program_id(0)

    @pl.when(i == 0)
    def _():
        acc_ref[...] = jnp.zeros_like(acc_ref)

    psT = pp_ref[0].T
    hsT = tab_ref[...].T[0:8]
    cnt = jnp.maximum(psT[32:33], 1.0)
    inv = 1.0 / cnt
    agg_s = psT[0:8] * inv
    V = [psT[8:16] * inv, psT[16:24] * inv, psT[24:32] * inv]

    def gvp_ln(sv, Vv, g, b):
        mu = jnp.mean(sv, axis=0, keepdims=True)
        var = jnp.mean((sv - mu) ** 2, axis=0, keepdims=True)
        sn = (sv - mu) / jnp.sqrt(var + 1e-5) * g + b
        nsq = jnp.maximum(Vv[0] ** 2 + Vv[1] ** 2 + Vv[2] ** 2, 1e-8)
        den = 1.0 / jnp.sqrt(jnp.mean(nsq, axis=0, keepdims=True))
        return sn, [v * den for v in Vv]

    def gvp(sv, Vv, wh, ws, wsb, wsv, wsvb, wv, act):
        H = [_dotT(wh, v) for v in Vv]
        vn = jnp.sqrt(jnp.maximum(H[0] ** 2 + H[1] ** 2 + H[2] ** 2, 1e-8))
        so = _dotT(ws, jnp.concatenate([sv, vn], axis=0)) + wsb
        gate = jax.nn.sigmoid(_dotT(wsv, so) + wsvb)
        Vo = [_dotT(wv, h) * gate for h in H]
        if act is not None:
            so = act(so)
        return so, Vo

    xs, Xv = gvp_ln(hsT + agg_s, V, ln0g_ref[...], ln0b_ref[...])
    fs1, Fv1 = gvp(xs, Xv, f1h_ref[...], f1w_ref[...], f1wb_ref[...],
                   f1gw_ref[...], f1gb_ref[...], f1v_ref[...], _silu)
    fs2, Fv2 = gvp(fs1, Fv1, f2h_ref[...], f2w_ref[...], f2wb_ref[...],
                   f2gw_ref[...], f2gb_ref[...], f2v_ref[...], None)
    ys, Yv = gvp_ln(xs + fs2, [Xv[d] + Fv2[d] for d in range(3)],
                    ln1g_ref[...], ln1b_ref[...])
    os_, Ov = gvp_ln(ys, Yv, lnog_ref[...], lnob_ref[...])
    OH = [_dotT(oh_ref[...], v) for v in Ov]
    vn3 = jnp.sqrt(jnp.maximum(OH[0] ** 2 + OH[1] ** 2 + OH[2] ** 2, 1e-8))
    out_sT = _dotT(ow_ref[...], jnp.concatenate([os_, vn3], axis=0)) \
        + owb_ref[...]

    bb = batch_ref[0]
    onehot = (lax.broadcasted_iota(jnp.int32, (NG, bb.shape[1]), 0)
              == bb).astype(jnp.float32)
    sums = lax.dot_general(out_sT, onehot, (((1,), (1,)), ((), ())),
                           preferred_element_type=jnp.float32)
    cnts = lax.dot_general(jnp.ones((1, bb.shape[1]), jnp.float32), onehot,
                           (((1,), (1,)), ((), ())),
                           preferred_element_type=jnp.float32)
    acc_ref[0:2, 0:NG] += sums
    acc_ref[2:3, 0:NG] += cnts

    @pl.when(i == nsteps - 1)
    def _():
        o_ref[...] = (acc_ref[0:2, 0:NG]
                      / jnp.maximum(acc_ref[2:3, 0:NG], 1.0)).T


def _node_out(table, partials, batch3d, p):
    f1, f2, go = p['ff1'], p['ff2'], p['gvp_out']
    nsteps = N // BN5
    blocks_per_core = NHALF // BN5
    return pl.pallas_call(
        functools.partial(_node_out_kernel, nsteps=nsteps),
        grid=(nsteps,),
        in_specs=[
            pl.BlockSpec((BN5, D_TAB), lambda i: (i, 0)),
            pl.BlockSpec((1, BN5, D_MSG),
                         lambda i: (i // blocks_per_core,
                                    i % blocks_per_core, 0)),
            pl.BlockSpec((1, 1, BN5), lambda i: (i, 0, 0)),
            _full((8, 1)), _full((8, 1)),
            _full((8, 16)), _full((24, 32)), _full((32, 1)), _full((32, 16)),
            _full((16, 1)), _full((16, 16)),
            _full((16, 16)), _full((48, 8)), _full((8, 1)), _full((8, 8)),
            _full((8, 1)), _full((16, 8)),
            _full((8, 1)), _full((8, 1)), _full((8, 1)), _full((8, 1)),
            _full((8, 8)), _full((16, 2)), _full((2, 1)),
        ],
        out_specs=pl.BlockSpec((NG, 2), lambda i: (0, 0)),
        out_shape=jax.ShapeDtypeStruct((NG, 2), jnp.float32),
        scratch_shapes=[pltpu.VMEM((8, 128), jnp.float32)],
    )(table, partials, batch3d,
      p['ln0_g'].reshape(8, 1), p['ln0_b'].reshape(8, 1),
      f1['wh'], f1['ws_w'], f1['ws_b'].reshape(32, 1), f1['wsv_w'],
      f1['wsv_b'].reshape(16, 1), f1['wv'],
      f2['wh'], f2['ws_w'], f2['ws_b'].reshape(8, 1), f2['wsv_w'],
      f2['wsv_b'].reshape(8, 1), f2['wv'],
      p['ln1_g'].reshape(8, 1), p['ln1_b'].reshape(8, 1),
      p['ln_out_g'].reshape(8, 1), p['ln_out_b'].reshape(8, 1),
      go['wh'], go['ws_w'], go['ws_b'].reshape(2, 1))


def kernel(positions, shifts, node_attrs, edge_index, batch, params):
    src = edge_index[0]
    dst = edge_index[1]
    pad = jnp.zeros((EP - E,), jnp.int32)
    src2d = jnp.concatenate([src, pad]).reshape(EP // 128, 128)
    dst2d = jnp.concatenate([dst, pad]).reshape(EP // 128, 128)

    table = _node_table(node_attrs, positions, params)
    srcf, dstf = _sc_gather(table, src2d, dst2d)
    msg = _edge_msg(srcf, dstf, params)
    partials = _sc_scatter(msg, dst2d)
    return _node_out(table, partials, batch.reshape(N // BN5, 1, BN5), params)


# speedup vs baseline: 43.9052x; 1.1334x over previous
import functools

import jax
import jax.numpy as jnp
from jax import lax
from jax.experimental import pallas as pl
from jax.experimental.pallas import tpu as pltpu
from jax.experimental.pallas import tpu_sc as plsc

N = 50000
E = 800000
EP = 819200
NG = 32
NB = 8
CUT = 5.0

BN = 2000
BN5 = 1000
BE = 8192
D_TAB = 16
D_MSG = 40
CH = 1024
KSUB = 8
ROWS_W = 200


def _silu(x):
    return x * jax.nn.sigmoid(x)


def _ln(s, g, b):
    mu = jnp.mean(s, axis=-1, keepdims=True)
    var = jnp.mean((s - mu) ** 2, axis=-1, keepdims=True)
    return (s - mu) / jnp.sqrt(var + 1e-5) * g + b


def _full(shape):
    return pl.BlockSpec(shape, lambda i: tuple(0 for _ in shape))


def _node_table_kernel(na_ref, pos_ref, g_ref, b_ref, w_ref, wb_ref, o_ref):
    sn = _ln(na_ref[...], g_ref[...], b_ref[...])
    hs = jnp.dot(sn, w_ref[...], preferred_element_type=jnp.float32) + wb_ref[...]
    o_ref[...] = jnp.concatenate(
        [hs, pos_ref[...], jnp.zeros((hs.shape[0], 5), jnp.float32)], axis=1)


def _node_table(node_attrs, positions, p):
    gv = p['gvp_v']
    return pl.pallas_call(
        _node_table_kernel,
        grid=(N // BN,),
        in_specs=[
            pl.BlockSpec((BN, 4), lambda i: (i, 0)),
            pl.BlockSpec((BN, 3), lambda i: (i, 0)),
            _full((1, 4)), _full((1, 4)), _full((4, 8)), _full((1, 8)),
        ],
        out_specs=pl.BlockSpec((BN, D_TAB), lambda i: (i, 0)),
        out_shape=jax.ShapeDtypeStruct((N, D_TAB), jnp.float32),
    )(node_attrs, positions,
      p['ln_v_g'].reshape(1, 4), p['ln_v_b'].reshape(1, 4),
      gv['ws_w'], gv['ws_b'].reshape(1, 8))


EH = EP // 2
GROWS = EH // 128
GROWS_W = GROWS // 32
KG = 5
CHG = KG * 128


def _sc_gather(table, src2d, dst2d):
    mesh = plsc.VectorSubcoreMesh(core_axis_name="c", subcore_axis_name="s")

    @functools.partial(
        pl.kernel,
        mesh=mesh,
        out_type=(jax.ShapeDtypeStruct((EH, D_TAB), jnp.float32),
                  jax.ShapeDtypeStruct((EH, D_TAB), jnp.float32)),
        scratch_types=[
            pltpu.VMEM((KG, 128), jnp.int32),
            pltpu.VMEM((CHG, D_TAB), jnp.float32),
            pltpu.SemaphoreType.DMA,
        ],
        compiler_params=pltpu.CompilerParams(use_tc_tiling_on_sc=False),
    )
    def k(tab_hbm, src_hbm, dst_hbm, osrc_hbm, odst_hbm, idx_v, rows_v, sem):
        w = lax.axis_index("c") * 16 + lax.axis_index("s")
        for i_hbm, o_hbm in ((src_hbm, osrc_hbm), (dst_hbm, odst_hbm)):
            @pl.loop(0, GROWS_W // KG)
            def _(t):
                r0 = w * GROWS_W + t * KG
                pltpu.sync_copy(i_hbm.at[pl.ds(r0, KG)], idx_v)
                cps = [
                    pltpu.async_copy(tab_hbm.at[idx_v.at[j]],
                                     rows_v.at[pl.ds(j * 128, 128)], sem)
                    for j in range(KG)
                ]
                for cp in cps:
                    cp.wait()
                pltpu.sync_copy(rows_v, o_hbm.at[pl.ds(r0 * 128, CHG)])

    return k(table, src2d, dst2d)


def _dotT(w, x):
    return lax.dot_general(w, x, (((0,), (0,)), ((), ())),
                           preferred_element_type=jnp.float32)


def _edge_msg_kernel(sf_ref, df_ref,
                     lneg_ref, lneb_ref, gew_ref, gewb_ref, gegw_ref,
                     scal_ref, wh8sq_ref, m1w_ref, m1wb_ref, m1gw_ref,
                     m1gb_ref, c1_ref, m2h_ref, m2w_ref, m2wb_ref,
                     m2gw_ref, m2gb_ref, m2v_ref, o_ref, *, base):
    i = pl.program_id(0)
    sfT = sf_ref[...][:, 0:11].T
    dfT = df_ref[...][:, 0:11].T
    a_e = scal_ref[0, 0]
    b_e = scal_ref[0, 1]
    ge_gate_b = scal_ref[0, 2]

    hss, pos_s = sfT[0:8], sfT[8:11]
    hsd, pos_d = dfT[0:8], dfT[8:11]
    vec = pos_d - pos_s
    lsq = jnp.maximum(jnp.sum(vec * vec, axis=0, keepdims=True), 1e-12)
    length = jnp.sqrt(lsq)
    inv_len = 1.0 / length
    unit = vec * inv_len
    wfreq = ((lax.broadcasted_iota(jnp.int32, (NB, 1), 0) + 1).astype(jnp.float32)
             * (jnp.pi / CUT))
    bess = jnp.sqrt(2.0 / CUT) * jnp.sin(length * wfreq) * inv_len
    u = length * (1.0 / CUT)
    u3 = u * u * u
    u6 = u3 * u3
    u7 = u6 * u
    u8 = u7 * u
    env = (1.0 - 28.0 * u6 + 48.0 * u7 - 21.0 * u8) * (length < CUT).astype(jnp.float32)
    edge_s = bess * env

    mu = jnp.mean(edge_s, axis=0, keepdims=True)
    var = jnp.mean((edge_s - mu) ** 2, axis=0, keepdims=True)
    es0 = (edge_s - mu) / jnp.sqrt(var + 1e-5) * lneg_ref[...] + lneb_ref[...]
    nsq = jnp.sum(unit * unit, axis=0, keepdims=True)
    ev0 = unit / jnp.sqrt(jnp.maximum(nsq, 1e-8))
    ev0sq = jnp.sum(ev0 * ev0, axis=0, keepdims=True)
    vn_e = jnp.sqrt(jnp.maximum(ev0sq * a_e, 1e-8))
    es = _dotT(gew_ref[...], jnp.concatenate([es0, vn_e], axis=0)) \
        + gewb_ref[...]
    gate_e = jax.nn.sigmoid(
        jnp.sum(es * gegw_ref[...], axis=0, keepdims=True) + ge_gate_b)
    evv = ev0 * (b_e * gate_e)
    evvsq = jnp.sum(evv * evv, axis=0, keepdims=True)

    vn1 = jnp.sqrt(jnp.maximum(evvsq * wh8sq_ref[...], 1e-8))
    x41 = jnp.concatenate([hss, es, hsd, vn1], axis=0)
    s1 = _dotT(m1w_ref[...], x41) + m1wb_ref[...]
    gate1 = jax.nn.sigmoid(_dotT(m1gw_ref[...], s1) + m1gb_ref[...])
    g1 = c1_ref[...] * gate1
    s1 = _silu(s1)

    h2 = _dotT(m2h_ref[...], g1)
    vn2 = jnp.sqrt(jnp.maximum(evvsq * h2 * h2, 1e-8))
    ms = _dotT(m2w_ref[...], jnp.concatenate([s1, vn2], axis=0)) \
        + m2wb_ref[...]
    gate2 = jax.nn.sigmoid(_dotT(m2gw_ref[...], ms) + m2gb_ref[...])
    g2 = _dotT(m2v_ref[...], h2) * gate2

    gid = lax.broadcasted_iota(jnp.int32, (1, sfT.shape[1]), 1) + i * BE + base
    valid = (gid < E).astype(jnp.float32)
    outT = jnp.concatenate(
        [ms, g2 * evv[0:1], g2 * evv[1:2], g2 * evv[2:3],
         valid, jnp.zeros((7, sfT.shape[1]), jnp.float32)], axis=0) * valid
    o_ref[...] = outT.T


def _edge_msg(srcf, dstf, p, base):
    ge, m1, m2 = p['gvp_e'], p['msg1'], p['msg2']
    wh8 = m1['wh'][8, :]
    scal = jnp.stack([ge['wh'][0, 0] ** 2,
                      ge['wh'][0, 0] * ge['wv'][0, 0],
                      ge['wsv_b'][0]]).reshape(1, 3)
    return pl.pallas_call(
        functools.partial(_edge_msg_kernel, base=base),
        grid=(EH // BE,),
        in_specs=[
            pl.BlockSpec((BE, D_TAB), lambda i: (i, 0)),
            pl.BlockSpec((BE, D_TAB), lambda i: (i, 0)),
            _full((8, 1)), _full((8, 1)), _full((9, 8)), _full((8, 1)),
            _full((8, 1)), _full((1, 3)), _full((17, 1)), _full((41, 8)),
            _full((8, 1)), _full((8, 8)), _full((8, 1)), _full((8, 1)),
            _full((8, 8)), _full((16, 8)), _full((8, 1)), _full((8, 8)),
            _full((8, 1)), _full((8, 8)),
        ],
        out_specs=pl.BlockSpec((BE, D_MSG), lambda i: (i, 0)),
        out_shape=jax.ShapeDtypeStruct((EH, D_MSG), jnp.float32),
    )(srcf, dstf,
      p['ln_e_g'].reshape(8, 1), p['ln_e_b'].reshape(8, 1),
      ge['ws_w'], ge['ws_b'].reshape(8, 1), ge['wsv_w'].reshape(8, 1),
      scal, (wh8 * wh8).reshape(17, 1),
      m1['ws_w'], m1['ws_b'].reshape(8, 1), m1['wsv_w'],
      m1['wsv_b'].reshape(8, 1), (wh8 @ m1['wv']).reshape(8, 1),
      m2['wh'], m2['ws_w'], m2['ws_b'].reshape(8, 1), m2['wsv_w'],
      m2['wsv_b'].reshape(8, 1), m2['wv'])


NHALF = N // 2
NACC = 26000
ZROWS = 325
SROWS = NACC // 16


def _sc_scatter(msg, dst2d):
    mesh = plsc.VectorSubcoreMesh(core_axis_name="c", subcore_axis_name="s")
    rows_per_sub = GROWS // 16

    @functools.partial(
        pl.kernel,
        mesh=mesh,
        out_type=jax.ShapeDtypeStruct((2, NACC, D_MSG), jnp.float32),
        scratch_types=[
            pltpu.VMEM((KSUB, 128), jnp.int32),
            pltpu.VMEM((CH, D_MSG), jnp.float32),
            pltpu.VMEM((ZROWS, D_MSG), jnp.float32),
            pltpu.VMEM_SHARED((NACC, D_MSG), jnp.float32),
        ],
        compiler_params=pltpu.CompilerParams(use_tc_tiling_on_sc=False),
    )
    def k(msg_hbm, dst_hbm, out_hbm, idx_v, msg_v, zbuf, acc):
        c = lax.axis_index("c")
        s = lax.axis_index("s")
        lo = c * NHALF
        zero16 = jnp.zeros((16,), jnp.float32)

        @pl.loop(0, ZROWS)
        def _(r):
            zbuf[r, pl.ds(0, 16)] = zero16
            zbuf[r, pl.ds(16, 16)] = zero16
            zbuf[r, pl.ds(24, 16)] = zero16

        @pl.loop(0, 5)
        def _(q):
            pltpu.sync_copy(zbuf, acc.at[pl.ds(s * SROWS + q * ZROWS, ZROWS)])

        plsc.subcore_barrier()

        @pl.loop(0, rows_per_sub // KSUB)
        def _(t):
            r0 = s * rows_per_sub + t * KSUB
            pltpu.sync_copy(dst_hbm.at[pl.ds(r0, KSUB)], idx_v)
            pltpu.sync_copy(msg_hbm.at[pl.ds(r0 * 128, CH)], msg_v)

            @pl.loop(0, KSUB)
            def _(r):
                @pl.loop(0, 8)
                def _(kk):
                    v = idx_v[r, pl.ds(kk * 16, 16)] - lo
                    ok = (v >= 0) & (v < NHALF)
                    idx_v[r, pl.ds(kk * 16, 16)] = jnp.where(ok, v, NHALF)

            for j in range(KSUB):
                pltpu.sync_copy(msg_v.at[pl.ds(j * 128, 128)],
                                acc.at[idx_v.at[j]], add=True)

        plsc.subcore_barrier()

        @pl.loop(0, 5)
        def _(q):
            r = s * SROWS + q * ZROWS
            pltpu.sync_copy(acc.at[pl.ds(r, ZROWS)],
                            out_hbm.at[c, pl.ds(r, ZROWS)])

    return k(msg, dst2d)


def _node_out_kernel(tab_ref, pp_ref, pp2_ref, batch_ref,
                     ln0g_ref, ln0b_ref,
                     f1h_ref, f1w_ref, f1wb_ref, f1gw_ref, f1gb_ref, f1v_ref,
                     f2h_ref, f2w_ref, f2wb_ref, f2gw_ref, f2gb_ref, f2v_ref,
                     ln1g_ref, ln1b_ref, lnog_ref, lnob_ref,
                     oh_ref, ow_ref, owb_ref, o_ref, acc_ref, *, nsteps):
    i = pl.program_id(0)

    @pl.when(i == 0)
    def _():
        acc_ref[...] = jnp.zeros_like(acc_ref)

    psT = (pp_ref[0] + pp2_ref[0]).T
    hsT = tab_ref[...].T[0:8]
    cnt = jnp.maximum(psT[32:33], 1.0)
    inv = 1.0 / cnt
    agg_s = psT[0:8] * inv
    V = [psT[8:16] * inv, psT[16:24] * inv, psT[24:32] * inv]

    def gvp_ln(sv, Vv, g, b):
        mu = jnp.mean(sv, axis=0, keepdims=True)
        var = jnp.mean((sv - mu) ** 2, axis=0, keepdims=True)
        sn = (sv - mu) / jnp.sqrt(var + 1e-5) * g + b
        nsq = jnp.maximum(Vv[0] ** 2 + Vv[1] ** 2 + Vv[2] ** 2, 1e-8)
        den = 1.0 / jnp.sqrt(jnp.mean(nsq, axis=0, keepdims=True))
        return sn, [v * den for v in Vv]

    def gvp(sv, Vv, wh, ws, wsb, wsv, wsvb, wv, act):
        H = [_dotT(wh, v) for v in Vv]
        vn = jnp.sqrt(jnp.maximum(H[0] ** 2 + H[1] ** 2 + H[2] ** 2, 1e-8))
        so = _dotT(ws, jnp.concatenate([sv, vn], axis=0)) + wsb
        gate = jax.nn.sigmoid(_dotT(wsv, so) + wsvb)
        Vo = [_dotT(wv, h) * gate for h in H]
        if act is not None:
            so = act(so)
        return so, Vo

    xs, Xv = gvp_ln(hsT + agg_s, V, ln0g_ref[...], ln0b_ref[...])
    fs1, Fv1 = gvp(xs, Xv, f1h_ref[...], f1w_ref[...], f1wb_ref[...],
                   f1gw_ref[...], f1gb_ref[...], f1v_ref[...], _silu)
    fs2, Fv2 = gvp(fs1, Fv1, f2h_ref[...], f2w_ref[...], f2wb_ref[...],
                   f2gw_ref[...], f2gb_ref[...], f2v_ref[...], None)
    ys, Yv = gvp_ln(xs + fs2, [Xv[d] + Fv2[d] for d in range(3)],
                    ln1g_ref[...], ln1b_ref[...])
    os_, Ov = gvp_ln(ys, Yv, lnog_ref[...], lnob_ref[...])
    OH = [_dotT(oh_ref[...], v) for v in Ov]
    vn3 = jnp.sqrt(jnp.maximum(OH[0] ** 2 + OH[1] ** 2 + OH[2] ** 2, 1e-8))
    out_sT = _dotT(ow_ref[...], jnp.concatenate([os_, vn3], axis=0)) \
        + owb_ref[...]

    bb = batch_ref[0]
    onehot = (lax.broadcasted_iota(jnp.int32, (NG, bb.shape[1]), 0)
              == bb).astype(jnp.float32)
    sums = lax.dot_general(out_sT, onehot, (((1,), (1,)), ((), ())),
                           preferred_element_type=jnp.float32)
    cnts = lax.dot_general(jnp.ones((1, bb.shape[1]), jnp.float32), onehot,
                           (((1,), (1,)), ((), ())),
                           preferred_element_type=jnp.float32)
    acc_ref[0:2, 0:NG] += sums
    acc_ref[2:3, 0:NG] += cnts

    @pl.when(i == nsteps - 1)
    def _():
        o_ref[...] = (acc_ref[0:2, 0:NG]
                      / jnp.maximum(acc_ref[2:3, 0:NG], 1.0)).T


def _node_out(table, partials, partials2, batch3d, p):
    f1, f2, go = p['ff1'], p['ff2'], p['gvp_out']
    nsteps = N // BN5
    blocks_per_core = NHALF // BN5
    pspec = pl.BlockSpec((1, BN5, D_MSG),
                         lambda i: (i // blocks_per_core,
                                    i % blocks_per_core, 0))
    return pl.pallas_call(
        functools.partial(_node_out_kernel, nsteps=nsteps),
        grid=(nsteps,),
        in_specs=[
            pl.BlockSpec((BN5, D_TAB), lambda i: (i, 0)),
            pspec, pspec,
            pl.BlockSpec((1, 1, BN5), lambda i: (i, 0, 0)),
            _full((8, 1)), _full((8, 1)),
            _full((8, 16)), _full((24, 32)), _full((32, 1)), _full((32, 16)),
            _full((16, 1)), _full((16, 16)),
            _full((16, 16)), _full((48, 8)), _full((8, 1)), _full((8, 8)),
            _full((8, 1)), _full((16, 8)),
            _full((8, 1)), _full((8, 1)), _full((8, 1)), _full((8, 1)),
            _full((8, 8)), _full((16, 2)), _full((2, 1)),
        ],
        out_specs=pl.BlockSpec((NG, 2), lambda i: (0, 0)),
        out_shape=jax.ShapeDtypeStruct((NG, 2), jnp.float32),
        scratch_shapes=[pltpu.VMEM((8, 128), jnp.float32)],
    )(table, partials, partials2, batch3d,
      p['ln0_g'].reshape(8, 1), p['ln0_b'].reshape(8, 1),
      f1['wh'], f1['ws_w'], f1['ws_b'].reshape(32, 1), f1['wsv_w'],
      f1['wsv_b'].reshape(16, 1), f1['wv'],
      f2['wh'], f2['ws_w'], f2['ws_b'].reshape(8, 1), f2['wsv_w'],
      f2['wsv_b'].reshape(8, 1), f2['wv'],
      p['ln1_g'].reshape(8, 1), p['ln1_b'].reshape(8, 1),
      p['ln_out_g'].reshape(8, 1), p['ln_out_b'].reshape(8, 1),
      go['wh'], go['ws_w'], go['ws_b'].reshape(2, 1))


def kernel(positions, shifts, node_attrs, edge_index, batch, params):
    src = edge_index[0]
    dst = edge_index[1]
    pad = jnp.zeros((EP - E,), jnp.int32)
    src2d = jnp.concatenate([src, pad]).reshape(EP // 128, 128)
    dst2d = jnp.concatenate([dst, pad]).reshape(EP // 128, 128)

    table = _node_table(node_attrs, positions, params)
    parts = []
    for h in range(2):
        s2 = src2d[h * GROWS:(h + 1) * GROWS]
        d2 = dst2d[h * GROWS:(h + 1) * GROWS]
        sf, df = _sc_gather(table, s2, d2)
        msg = _edge_msg(sf, df, params, h * EH)
        parts.append(_sc_scatter(msg, d2))
    return _node_out(table, parts[0], parts[1],
                     batch.reshape(N // BN5, 1, BN5), params)


# speedup vs baseline: 74.7229x; 1.7019x over previous
import functools

import jax
import jax.numpy as jnp
from jax import lax
from jax.experimental import pallas as pl
from jax.experimental.pallas import tpu as pltpu
from jax.experimental.pallas import tpu_sc as plsc

N = 50000
E = 800000
EP = 819200
NG = 32
NB = 8
CUT = 5.0

BN = 2000
BN5 = 1000
BE = 8192
D_TAB = 16
D_MSG = 40
CH = 1024
KSUB = 8
ROWS_W = 200


def _silu(x):
    return x * jax.nn.sigmoid(x)


def _ln(s, g, b):
    mu = jnp.mean(s, axis=-1, keepdims=True)
    var = jnp.mean((s - mu) ** 2, axis=-1, keepdims=True)
    return (s - mu) / jnp.sqrt(var + 1e-5) * g + b


def _full(shape):
    return pl.BlockSpec(shape, lambda i: tuple(0 for _ in shape))


def _node_table_kernel(na_ref, pos_ref, g_ref, b_ref, w_ref, wb_ref, o_ref):
    sn = _ln(na_ref[...], g_ref[...], b_ref[...])
    hs = jnp.dot(sn, w_ref[...], preferred_element_type=jnp.float32) + wb_ref[...]
    o_ref[...] = jnp.concatenate(
        [hs, pos_ref[...], jnp.zeros((hs.shape[0], 5), jnp.float32)], axis=1)


def _node_table(node_attrs, positions, p):
    gv = p['gvp_v']
    return pl.pallas_call(
        _node_table_kernel,
        grid=(N // BN,),
        in_specs=[
            pl.BlockSpec((BN, 4), lambda i: (i, 0)),
            pl.BlockSpec((BN, 3), lambda i: (i, 0)),
            _full((1, 4)), _full((1, 4)), _full((4, 8)), _full((1, 8)),
        ],
        out_specs=pl.BlockSpec((BN, D_TAB), lambda i: (i, 0)),
        out_shape=jax.ShapeDtypeStruct((N, D_TAB), jnp.float32),
    )(node_attrs, positions,
      p['ln_v_g'].reshape(1, 4), p['ln_v_b'].reshape(1, 4),
      gv['ws_w'], gv['ws_b'].reshape(1, 8))


EH = EP // 2
GROWS = EH // 128
GROWS_W = GROWS // 32
KG = 5
CHG = KG * 128


def _sc_gather(table, src2d, dst2d):
    mesh = plsc.VectorSubcoreMesh(core_axis_name="c", subcore_axis_name="s")

    @functools.partial(
        pl.kernel,
        mesh=mesh,
        out_type=(jax.ShapeDtypeStruct((EH, D_TAB), jnp.float32),
                  jax.ShapeDtypeStruct((EH, D_TAB), jnp.float32)),
        scratch_types=[
            pltpu.VMEM((KG, 128), jnp.int32),
            pltpu.VMEM((CHG, D_TAB), jnp.float32),
            pltpu.SemaphoreType.DMA,
        ],
        compiler_params=pltpu.CompilerParams(use_tc_tiling_on_sc=False),
    )
    def k(tab_hbm, src_hbm, dst_hbm, osrc_hbm, odst_hbm, idx_v, rows_v, sem):
        w = lax.axis_index("c") * 16 + lax.axis_index("s")
        for i_hbm, o_hbm in ((src_hbm, osrc_hbm), (dst_hbm, odst_hbm)):
            @pl.loop(0, GROWS_W // KG)
            def _(t):
                r0 = w * GROWS_W + t * KG
                pltpu.sync_copy(i_hbm.at[pl.ds(r0, KG)], idx_v)
                cps = [
                    pltpu.async_copy(tab_hbm.at[idx_v.at[j]],
                                     rows_v.at[pl.ds(j * 128, 128)], sem)
                    for j in range(KG)
                ]
                for cp in cps:
                    cp.wait()
                pltpu.sync_copy(rows_v, o_hbm.at[pl.ds(r0 * 128, CHG)])

    return k(table, src2d, dst2d)


def _dotT(w, x):
    return lax.dot_general(w, x, (((0,), (0,)), ((), ())),
                           preferred_element_type=jnp.float32)


def _edge_msg_kernel(sf_ref, df_ref,
                     lneg_ref, lneb_ref, gew_ref, gewb_ref, gegw_ref,
                     scal_ref, wh8sq_ref, m1w_ref, m1wb_ref, m1gw_ref,
                     m1gb_ref, c1_ref, m2h_ref, m2w_ref, m2wb_ref,
                     m2gw_ref, m2gb_ref, m2v_ref, o_ref, *, base):
    i = pl.program_id(0)

    def unpack(ref):
        xt = ref[...].T
        return jnp.concatenate(
            [xt[16 * j:16 * j + 16, :] for j in range(8)], axis=1)

    sfT = unpack(sf_ref)[0:11]
    dfT = unpack(df_ref)[0:11]
    a_e = scal_ref[0, 0]
    b_e = scal_ref[0, 1]
    ge_gate_b = scal_ref[0, 2]

    hss, pos_s = sfT[0:8], sfT[8:11]
    hsd, pos_d = dfT[0:8], dfT[8:11]
    vec = pos_d - pos_s
    lsq = jnp.maximum(jnp.sum(vec * vec, axis=0, keepdims=True), 1e-12)
    length = jnp.sqrt(lsq)
    inv_len = 1.0 / length
    unit = vec * inv_len
    wfreq = ((lax.broadcasted_iota(jnp.int32, (NB, 1), 0) + 1).astype(jnp.float32)
             * (jnp.pi / CUT))
    bess = jnp.sqrt(2.0 / CUT) * jnp.sin(length * wfreq) * inv_len
    u = length * (1.0 / CUT)
    u3 = u * u * u
    u6 = u3 * u3
    u7 = u6 * u
    u8 = u7 * u
    env = (1.0 - 28.0 * u6 + 48.0 * u7 - 21.0 * u8) * (length < CUT).astype(jnp.float32)
    edge_s = bess * env

    mu = jnp.mean(edge_s, axis=0, keepdims=True)
    var = jnp.mean((edge_s - mu) ** 2, axis=0, keepdims=True)
    es0 = (edge_s - mu) / jnp.sqrt(var + 1e-5) * lneg_ref[...] + lneb_ref[...]
    nsq = jnp.sum(unit * unit, axis=0, keepdims=True)
    ev0 = unit / jnp.sqrt(jnp.maximum(nsq, 1e-8))
    ev0sq = jnp.sum(ev0 * ev0, axis=0, keepdims=True)
    vn_e = jnp.sqrt(jnp.maximum(ev0sq * a_e, 1e-8))
    es = _dotT(gew_ref[...], jnp.concatenate([es0, vn_e], axis=0)) \
        + gewb_ref[...]
    gate_e = jax.nn.sigmoid(
        jnp.sum(es * gegw_ref[...], axis=0, keepdims=True) + ge_gate_b)
    evv = ev0 * (b_e * gate_e)
    evvsq = jnp.sum(evv * evv, axis=0, keepdims=True)

    vn1 = jnp.sqrt(jnp.maximum(evvsq * wh8sq_ref[...], 1e-8))
    x41 = jnp.concatenate([hss, es, hsd, vn1], axis=0)
    s1 = _dotT(m1w_ref[...], x41) + m1wb_ref[...]
    gate1 = jax.nn.sigmoid(_dotT(m1gw_ref[...], s1) + m1gb_ref[...])
    g1 = c1_ref[...] * gate1
    s1 = _silu(s1)

    h2 = _dotT(m2h_ref[...], g1)
    vn2 = jnp.sqrt(jnp.maximum(evvsq * h2 * h2, 1e-8))
    ms = _dotT(m2w_ref[...], jnp.concatenate([s1, vn2], axis=0)) \
        + m2wb_ref[...]
    gate2 = jax.nn.sigmoid(_dotT(m2gw_ref[...], ms) + m2gb_ref[...])
    g2 = _dotT(m2v_ref[...], h2) * gate2

    gid = lax.broadcasted_iota(jnp.int32, (1, BE), 1) + i * BE + base
    valid = (gid < E).astype(jnp.float32)
    outT = jnp.concatenate(
        [ms, g2 * evv[0:1], g2 * evv[1:2], g2 * evv[2:3],
         valid, jnp.zeros((7, BE), jnp.float32)], axis=0) * valid
    o_ref[:, 0:D_MSG] = outT.T


def _edge_msg(srcf, dstf, p, base):
    ge, m1, m2 = p['gvp_e'], p['msg1'], p['msg2']
    wh8 = m1['wh'][8, :]
    scal = jnp.stack([ge['wh'][0, 0] ** 2,
                      ge['wh'][0, 0] * ge['wv'][0, 0],
                      ge['wsv_b'][0]]).reshape(1, 3)
    return pl.pallas_call(
        functools.partial(_edge_msg_kernel, base=base),
        grid=(EH // BE,),
        in_specs=[
            pl.BlockSpec((BE * D_TAB // 128, 128), lambda i: (i, 0)),
            pl.BlockSpec((BE * D_TAB // 128, 128), lambda i: (i, 0)),
            _full((8, 1)), _full((8, 1)), _full((9, 8)), _full((8, 1)),
            _full((8, 1)), _full((1, 3)), _full((17, 1)), _full((41, 8)),
            _full((8, 1)), _full((8, 8)), _full((8, 1)), _full((8, 1)),
            _full((8, 8)), _full((16, 8)), _full((8, 1)), _full((8, 8)),
            _full((8, 1)), _full((8, 8)),
        ],
        out_specs=pl.BlockSpec((BE, 128), lambda i: (i, 0)),
        out_shape=jax.ShapeDtypeStruct((EH, 128), jnp.float32),
    )(srcf.reshape(EH * D_TAB // 128, 128),
      dstf.reshape(EH * D_TAB // 128, 128),
      p['ln_e_g'].reshape(8, 1), p['ln_e_b'].reshape(8, 1),
      ge['ws_w'], ge['ws_b'].reshape(8, 1), ge['wsv_w'].reshape(8, 1),
      scal, (wh8 * wh8).reshape(17, 1),
      m1['ws_w'], m1['ws_b'].reshape(8, 1), m1['wsv_w'],
      m1['wsv_b'].reshape(8, 1), (wh8 @ m1['wv']).reshape(8, 1),
      m2['wh'], m2['ws_w'], m2['ws_b'].reshape(8, 1), m2['wsv_w'],
      m2['wsv_b'].reshape(8, 1), m2['wv'])


NHALF = N // 2
NACC = 26000
ZROWS = 325
SROWS = NACC // 16


def _sc_scatter(msg, dst2d):
    mesh = plsc.VectorSubcoreMesh(core_axis_name="c", subcore_axis_name="s")
    rows_per_sub = GROWS // 16

    @functools.partial(
        pl.kernel,
        mesh=mesh,
        out_type=jax.ShapeDtypeStruct((2, NACC, D_MSG), jnp.float32),
        scratch_types=[
            pltpu.VMEM((KSUB, 128), jnp.int32),
            pltpu.VMEM((CH, D_MSG), jnp.float32),
            pltpu.VMEM((ZROWS, D_MSG), jnp.float32),
            pltpu.VMEM_SHARED((NACC, D_MSG), jnp.float32),
        ],
        compiler_params=pltpu.CompilerParams(use_tc_tiling_on_sc=False),
    )
    def k(msg_hbm, dst_hbm, out_hbm, idx_v, msg_v, zbuf, acc):
        c = lax.axis_index("c")
        s = lax.axis_index("s")
        lo = c * NHALF
        zero16 = jnp.zeros((16,), jnp.float32)

        @pl.loop(0, ZROWS)
        def _(r):
            zbuf[r, pl.ds(0, 16)] = zero16
            zbuf[r, pl.ds(16, 16)] = zero16
            zbuf[r, pl.ds(24, 16)] = zero16

        @pl.loop(0, 5)
        def _(q):
            pltpu.sync_copy(zbuf, acc.at[pl.ds(s * SROWS + q * ZROWS, ZROWS)])

        plsc.subcore_barrier()

        @pl.loop(0, rows_per_sub // KSUB)
        def _(t):
            r0 = s * rows_per_sub + t * KSUB
            pltpu.sync_copy(dst_hbm.at[pl.ds(r0, KSUB)], idx_v)
            pltpu.sync_copy(msg_hbm.at[pl.ds(r0 * 128, CH), pl.ds(0, D_MSG)],
                            msg_v)

            @pl.loop(0, KSUB)
            def _(r):
                @pl.loop(0, 8)
                def _(kk):
                    v = idx_v[r, pl.ds(kk * 16, 16)] - lo
                    ok = (v >= 0) & (v < NHALF)
                    idx_v[r, pl.ds(kk * 16, 16)] = jnp.where(ok, v, NHALF)

            for j in range(KSUB):
                pltpu.sync_copy(msg_v.at[pl.ds(j * 128, 128)],
                                acc.at[idx_v.at[j]], add=True)

        plsc.subcore_barrier()

        @pl.loop(0, 5)
        def _(q):
            r = s * SROWS + q * ZROWS
            pltpu.sync_copy(acc.at[pl.ds(r, ZROWS)],
                            out_hbm.at[c, pl.ds(r, ZROWS)])

    return k(msg, dst2d)


def _node_out_kernel(tab_ref, pp_ref, pp2_ref, batch_ref,
                     ln0g_ref, ln0b_ref,
                     f1h_ref, f1w_ref, f1wb_ref, f1gw_ref, f1gb_ref, f1v_ref,
                     f2h_ref, f2w_ref, f2wb_ref, f2gw_ref, f2gb_ref, f2v_ref,
                     ln1g_ref, ln1b_ref, lnog_ref, lnob_ref,
                     oh_ref, ow_ref, owb_ref, o_ref, acc_ref, *, nsteps):
    i = pl.program_id(0)

    @pl.when(i == 0)
    def _():
        acc_ref[...] = jnp.zeros_like(acc_ref)

    psT = (pp_ref[0] + pp2_ref[0]).T
    hsT = tab_ref[...].T[0:8]
    cnt = jnp.maximum(psT[32:33], 1.0)
    inv = 1.0 / cnt
    agg_s = psT[0:8] * inv
    V = [psT[8:16] * inv, psT[16:24] * inv, psT[24:32] * inv]

    def gvp_ln(sv, Vv, g, b):
        mu = jnp.mean(sv, axis=0, keepdims=True)
        var = jnp.mean((sv - mu) ** 2, axis=0, keepdims=True)
        sn = (sv - mu) / jnp.sqrt(var + 1e-5) * g + b
        nsq = jnp.maximum(Vv[0] ** 2 + Vv[1] ** 2 + Vv[2] ** 2, 1e-8)
        den = 1.0 / jnp.sqrt(jnp.mean(nsq, axis=0, keepdims=True))
        return sn, [v * den for v in Vv]

    def gvp(sv, Vv, wh, ws, wsb, wsv, wsvb, wv, act):
        H = [_dotT(wh, v) for v in Vv]
        vn = jnp.sqrt(jnp.maximum(H[0] ** 2 + H[1] ** 2 + H[2] ** 2, 1e-8))
        so = _dotT(ws, jnp.concatenate([sv, vn], axis=0)) + wsb
        gate = jax.nn.sigmoid(_dotT(wsv, so) + wsvb)
        Vo = [_dotT(wv, h) * gate for h in H]
        if act is not None:
            so = act(so)
        return so, Vo

    xs, Xv = gvp_ln(hsT + agg_s, V, ln0g_ref[...], ln0b_ref[...])
    fs1, Fv1 = gvp(xs, Xv, f1h_ref[...], f1w_ref[...], f1wb_ref[...],
                   f1gw_ref[...], f1gb_ref[...], f1v_ref[...], _silu)
    fs2, Fv2 = gvp(fs1, Fv1, f2h_ref[...], f2w_ref[...], f2wb_ref[...],
                   f2gw_ref[...], f2gb_ref[...], f2v_ref[...], None)
    ys, Yv = gvp_ln(xs + fs2, [Xv[d] + Fv2[d] for d in range(3)],
                    ln1g_ref[...], ln1b_ref[...])
    os_, Ov = gvp_ln(ys, Yv, lnog_ref[...], lnob_ref[...])
    OH = [_dotT(oh_ref[...], v) for v in Ov]
    vn3 = jnp.sqrt(jnp.maximum(OH[0] ** 2 + OH[1] ** 2 + OH[2] ** 2, 1e-8))
    out_sT = _dotT(ow_ref[...], jnp.concatenate([os_, vn3], axis=0)) \
        + owb_ref[...]

    bb = batch_ref[0]
    onehot = (lax.broadcasted_iota(jnp.int32, (NG, bb.shape[1]), 0)
              == bb).astype(jnp.float32)
    sums = lax.dot_general(out_sT, onehot, (((1,), (1,)), ((), ())),
                           preferred_element_type=jnp.float32)
    cnts = lax.dot_general(jnp.ones((1, bb.shape[1]), jnp.float32), onehot,
                           (((1,), (1,)), ((), ())),
                           preferred_element_type=jnp.float32)
    acc_ref[0:2, 0:NG] += sums
    acc_ref[2:3, 0:NG] += cnts

    @pl.when(i == nsteps - 1)
    def _():
        o_ref[...] = (acc_ref[0:2, 0:NG]
                      / jnp.maximum(acc_ref[2:3, 0:NG], 1.0)).T


def _node_out(table, partials, partials2, batch3d, p):
    f1, f2, go = p['ff1'], p['ff2'], p['gvp_out']
    nsteps = N // BN5
    blocks_per_core = NHALF // BN5
    pspec = pl.BlockSpec((1, BN5, D_MSG),
                         lambda i: (i // blocks_per_core,
                                    i % blocks_per_core, 0))
    return pl.pallas_call(
        functools.partial(_node_out_kernel, nsteps=nsteps),
        grid=(nsteps,),
        in_specs=[
            pl.BlockSpec((BN5, D_TAB), lambda i: (i, 0)),
            pspec, pspec,
            pl.BlockSpec((1, 1, BN5), lambda i: (i, 0, 0)),
            _full((8, 1)), _full((8, 1)),
            _full((8, 16)), _full((24, 32)), _full((32, 1)), _full((32, 16)),
            _full((16, 1)), _full((16, 16)),
            _full((16, 16)), _full((48, 8)), _full((8, 1)), _full((8, 8)),
            _full((8, 1)), _full((16, 8)),
            _full((8, 1)), _full((8, 1)), _full((8, 1)), _full((8, 1)),
            _full((8, 8)), _full((16, 2)), _full((2, 1)),
        ],
        out_specs=pl.BlockSpec((NG, 2), lambda i: (0, 0)),
        out_shape=jax.ShapeDtypeStruct((NG, 2), jnp.float32),
        scratch_shapes=[pltpu.VMEM((8, 128), jnp.float32)],
    )(table, partials, partials2, batch3d,
      p['ln0_g'].reshape(8, 1), p['ln0_b'].reshape(8, 1),
      f1['wh'], f1['ws_w'], f1['ws_b'].reshape(32, 1), f1['wsv_w'],
      f1['wsv_b'].reshape(16, 1), f1['wv'],
      f2['wh'], f2['ws_w'], f2['ws_b'].reshape(8, 1), f2['wsv_w'],
      f2['wsv_b'].reshape(8, 1), f2['wv'],
      p['ln1_g'].reshape(8, 1), p['ln1_b'].reshape(8, 1),
      p['ln_out_g'].reshape(8, 1), p['ln_out_b'].reshape(8, 1),
      go['wh'], go['ws_w'], go['ws_b'].reshape(2, 1))


def kernel(positions, shifts, node_attrs, edge_index, batch, params):
    src = edge_index[0]
    dst = edge_index[1]
    pad = jnp.zeros((EP - E,), jnp.int32)
    src2d = jnp.concatenate([src, pad]).reshape(EP // 128, 128)
    dst2d = jnp.concatenate([dst, pad]).reshape(EP // 128, 128)

    table = _node_table(node_attrs, positions, params)

    def perm(a2d):
        return (a2d.reshape(-1, 8, 1024).transpose(0, 2, 1)
                .reshape(GROWS, 128))

    parts = []
    for h in range(2):
        s2 = src2d[h * GROWS:(h + 1) * GROWS]
        d2 = dst2d[h * GROWS:(h + 1) * GROWS]
        sf, df = _sc_gather(table, perm(s2), perm(d2))
        msg = _edge_msg(sf, df, params, h * EH)
        parts.append(_sc_scatter(msg, d2))
    return _node_out(table, parts[0], parts[1],
                     batch.reshape(N // BN5, 1, BN5), params)


# speedup vs baseline: 77.1925x; 1.0330x over previous
import functools

import jax
import jax.numpy as jnp
from jax import lax
from jax.experimental import pallas as pl
from jax.experimental.pallas import tpu as pltpu
from jax.experimental.pallas import tpu_sc as plsc

N = 50000
E = 800000
EP = 819200
NG = 32
NB = 8
CUT = 5.0

BN = 2000
BN5 = 1000
BE = 8192
D_TAB = 16
D_MSG = 40
CH = 1024
KSUB = 8
ROWS_W = 200


def _silu(x):
    return x * jax.nn.sigmoid(x)


def _ln(s, g, b):
    mu = jnp.mean(s, axis=-1, keepdims=True)
    var = jnp.mean((s - mu) ** 2, axis=-1, keepdims=True)
    return (s - mu) / jnp.sqrt(var + 1e-5) * g + b


def _full(shape):
    return pl.BlockSpec(shape, lambda i: tuple(0 for _ in shape))


def _node_table_kernel(na_ref, pos_ref, g_ref, b_ref, w_ref, wb_ref, o_ref):
    sn = _ln(na_ref[...], g_ref[...], b_ref[...])
    hs = jnp.dot(sn, w_ref[...], preferred_element_type=jnp.float32) + wb_ref[...]
    o_ref[...] = jnp.concatenate(
        [hs, pos_ref[...], jnp.zeros((hs.shape[0], 5), jnp.float32)], axis=1)


def _node_table(node_attrs, positions, p):
    gv = p['gvp_v']
    return pl.pallas_call(
        _node_table_kernel,
        grid=(N // BN,),
        in_specs=[
            pl.BlockSpec((BN, 4), lambda i: (i, 0)),
            pl.BlockSpec((BN, 3), lambda i: (i, 0)),
            _full((1, 4)), _full((1, 4)), _full((4, 8)), _full((1, 8)),
        ],
        out_specs=pl.BlockSpec((BN, D_TAB), lambda i: (i, 0)),
        out_shape=jax.ShapeDtypeStruct((N, D_TAB), jnp.float32),
    )(node_attrs, positions,
      p['ln_v_g'].reshape(1, 4), p['ln_v_b'].reshape(1, 4),
      gv['ws_w'], gv['ws_b'].reshape(1, 8))


EH = EP // 2
GROWS = EH // 128
GROWS_W = GROWS // 32
KG = 5
CHG = KG * 128


def _sc_gather(table, src2d, dst2d):
    mesh = plsc.VectorSubcoreMesh(core_axis_name="c", subcore_axis_name="s")

    @functools.partial(
        pl.kernel,
        mesh=mesh,
        out_type=(jax.ShapeDtypeStruct((EH, D_TAB), jnp.float32),
                  jax.ShapeDtypeStruct((EH, D_TAB), jnp.float32)),
        scratch_types=[
            pltpu.VMEM((2, KG, 128), jnp.int32),
            pltpu.VMEM((2, CHG, D_TAB), jnp.float32),
            pltpu.SemaphoreType.DMA,
            pltpu.SemaphoreType.DMA,
            pltpu.SemaphoreType.DMA,
        ],
        compiler_params=pltpu.CompilerParams(use_tc_tiling_on_sc=False),
    )
    def k(tab_hbm, src_hbm, dst_hbm, osrc_hbm, odst_hbm, idx_v, rows_v,
          isem, gsem, wsem):
        w = lax.axis_index("c") * 16 + lax.axis_index("s")

        for i_hbm, o_hbm in ((src_hbm, osrc_hbm), (dst_hbm, odst_hbm)):
            def idx_cp(t, b):
                return pltpu.make_async_copy(
                    i_hbm.at[pl.ds(w * GROWS_W + t * KG, KG)],
                    idx_v.at[b], isem)

            def wb_cp(t, b):
                return pltpu.make_async_copy(
                    rows_v.at[b],
                    o_hbm.at[pl.ds((w * GROWS_W + t * KG) * 128, CHG)], wsem)

            idx_cp(0, 0).start()

            @pl.loop(0, GROWS_W // KG, step=2)
            def _(t):
                for half, b in ((0, 0), (1, 1)):
                    tt = t + half

                    @pl.when(tt > 1)
                    def _():
                        wb_cp(tt, b).wait()

                    idx_cp(tt, b).wait()

                    @pl.when(tt + 1 < GROWS_W // KG)
                    def _():
                        idx_cp(tt + 1, 1 - b).start()

                    cps = [
                        pltpu.async_copy(tab_hbm.at[idx_v.at[b, j]],
                                         rows_v.at[b, pl.ds(j * 128, 128)],
                                         gsem)
                        for j in range(KG)
                    ]
                    for cp in cps:
                        cp.wait()
                    wb_cp(tt, b).start()

            wb_cp(0, 0).wait()
            wb_cp(1, 1).wait()

    return k(table, src2d, dst2d)


def _dotT(w, x):
    return lax.dot_general(w, x, (((0,), (0,)), ((), ())),
                           preferred_element_type=jnp.float32)


def _edge_msg_kernel(sf_ref, df_ref,
                     lneg_ref, lneb_ref, gew_ref, gewb_ref, gegw_ref,
                     scal_ref, wh8sq_ref, m1w_ref, m1wb_ref, m1gw_ref,
                     m1gb_ref, c1_ref, m2h_ref, m2w_ref, m2wb_ref,
                     m2gw_ref, m2gb_ref, m2v_ref, o_ref, *, base):
    i = pl.program_id(0)

    def unpack(ref):
        xt = ref[...].T
        return jnp.concatenate(
            [xt[16 * j:16 * j + 16, :] for j in range(8)], axis=1)

    sfT = unpack(sf_ref)[0:11]
    dfT = unpack(df_ref)[0:11]
    a_e = scal_ref[0, 0]
    b_e = scal_ref[0, 1]
    ge_gate_b = scal_ref[0, 2]

    hss, pos_s = sfT[0:8], sfT[8:11]
    hsd, pos_d = dfT[0:8], dfT[8:11]
    vec = pos_d - pos_s
    lsq = jnp.maximum(jnp.sum(vec * vec, axis=0, keepdims=True), 1e-12)
    length = jnp.sqrt(lsq)
    inv_len = 1.0 / length
    unit = vec * inv_len
    wfreq = ((lax.broadcasted_iota(jnp.int32, (NB, 1), 0) + 1).astype(jnp.float32)
             * (jnp.pi / CUT))
    bess = jnp.sqrt(2.0 / CUT) * jnp.sin(length * wfreq) * inv_len
    u = length * (1.0 / CUT)
    u3 = u * u * u
    u6 = u3 * u3
    u7 = u6 * u
    u8 = u7 * u
    env = (1.0 - 28.0 * u6 + 48.0 * u7 - 21.0 * u8) * (length < CUT).astype(jnp.float32)
    edge_s = bess * env

    mu = jnp.mean(edge_s, axis=0, keepdims=True)
    var = jnp.mean((edge_s - mu) ** 2, axis=0, keepdims=True)
    es0 = (edge_s - mu) / jnp.sqrt(var + 1e-5) * lneg_ref[...] + lneb_ref[...]
    nsq = jnp.sum(unit * unit, axis=0, keepdims=True)
    ev0 = unit / jnp.sqrt(jnp.maximum(nsq, 1e-8))
    ev0sq = jnp.sum(ev0 * ev0, axis=0, keepdims=True)
    vn_e = jnp.sqrt(jnp.maximum(ev0sq * a_e, 1e-8))
    es = _dotT(gew_ref[...], jnp.concatenate([es0, vn_e], axis=0)) \
        + gewb_ref[...]
    gate_e = jax.nn.sigmoid(
        jnp.sum(es * gegw_ref[...], axis=0, keepdims=True) + ge_gate_b)
    evv = ev0 * (b_e * gate_e)
    evvsq = jnp.sum(evv * evv, axis=0, keepdims=True)

    vn1 = jnp.sqrt(jnp.maximum(evvsq * wh8sq_ref[...], 1e-8))
    x41 = jnp.concatenate([hss, es, hsd, vn1], axis=0)
    s1 = _dotT(m1w_ref[...], x41) + m1wb_ref[...]
    gate1 = jax.nn.sigmoid(_dotT(m1gw_ref[...], s1) + m1gb_ref[...])
    g1 = c1_ref[...] * gate1
    s1 = _silu(s1)

    h2 = _dotT(m2h_ref[...], g1)
    vn2 = jnp.sqrt(jnp.maximum(evvsq * h2 * h2, 1e-8))
    ms = _dotT(m2w_ref[...], jnp.concatenate([s1, vn2], axis=0)) \
        + m2wb_ref[...]
    gate2 = jax.nn.sigmoid(_dotT(m2gw_ref[...], ms) + m2gb_ref[...])
    g2 = _dotT(m2v_ref[...], h2) * gate2

    gid = lax.broadcasted_iota(jnp.int32, (1, BE), 1) + i * BE + base
    valid = (gid < E).astype(jnp.float32)
    outT = jnp.concatenate(
        [ms, g2 * evv[0:1], g2 * evv[1:2], g2 * evv[2:3],
         valid, jnp.zeros((7, BE), jnp.float32)], axis=0) * valid
    o_ref[:, 0:D_MSG] = outT.T


def _edge_msg(srcf, dstf, p, base):
    ge, m1, m2 = p['gvp_e'], p['msg1'], p['msg2']
    wh8 = m1['wh'][8, :]
    scal = jnp.stack([ge['wh'][0, 0] ** 2,
                      ge['wh'][0, 0] * ge['wv'][0, 0],
                      ge['wsv_b'][0]]).reshape(1, 3)
    return pl.pallas_call(
        functools.partial(_edge_msg_kernel, base=base),
        grid=(EH // BE,),
        in_specs=[
            pl.BlockSpec((BE * D_TAB // 128, 128), lambda i: (i, 0)),
            pl.BlockSpec((BE * D_TAB // 128, 128), lambda i: (i, 0)),
            _full((8, 1)), _full((8, 1)), _full((9, 8)), _full((8, 1)),
            _full((8, 1)), _full((1, 3)), _full((17, 1)), _full((41, 8)),
            _full((8, 1)), _full((8, 8)), _full((8, 1)), _full((8, 1)),
            _full((8, 8)), _full((16, 8)), _full((8, 1)), _full((8, 8)),
            _full((8, 1)), _full((8, 8)),
        ],
        out_specs=pl.BlockSpec((BE, 128), lambda i: (i, 0)),
        out_shape=jax.ShapeDtypeStruct((EH, 128), jnp.float32),
    )(srcf.reshape(EH * D_TAB // 128, 128),
      dstf.reshape(EH * D_TAB // 128, 128),
      p['ln_e_g'].reshape(8, 1), p['ln_e_b'].reshape(8, 1),
      ge['ws_w'], ge['ws_b'].reshape(8, 1), ge['wsv_w'].reshape(8, 1),
      scal, (wh8 * wh8).reshape(17, 1),
      m1['ws_w'], m1['ws_b'].reshape(8, 1), m1['wsv_w'],
      m1['wsv_b'].reshape(8, 1), (wh8 @ m1['wv']).reshape(8, 1),
      m2['wh'], m2['ws_w'], m2['ws_b'].reshape(8, 1), m2['wsv_w'],
      m2['wsv_b'].reshape(8, 1), m2['wv'])


NHALF = N // 2
NACC = 26000
ZROWS = 325
SROWS = NACC // 16


def _sc_scatter(msg, dst2d):
    mesh = plsc.VectorSubcoreMesh(core_axis_name="c", subcore_axis_name="s")
    rows_per_sub = GROWS // 16

    KS2 = 4
    CHS = KS2 * 128
    nchunk = rows_per_sub // KS2

    @functools.partial(
        pl.kernel,
        mesh=mesh,
        out_type=jax.ShapeDtypeStruct((2, NACC, D_MSG), jnp.float32),
        scratch_types=[
            pltpu.VMEM((2, KS2, 128), jnp.int32),
            pltpu.VMEM((2, CHS, D_MSG), jnp.float32),
            pltpu.VMEM((ZROWS, D_MSG), jnp.float32),
            pltpu.VMEM_SHARED((NACC, D_MSG), jnp.float32),
            pltpu.SemaphoreType.DMA,
            pltpu.SemaphoreType.DMA,
        ],
        compiler_params=pltpu.CompilerParams(use_tc_tiling_on_sc=False),
    )
    def k(msg_hbm, dst_hbm, out_hbm, idx_v, msg_v, zbuf, acc, lsem, asem):
        c = lax.axis_index("c")
        s = lax.axis_index("s")
        lo = c * NHALF
        zero16 = jnp.zeros((16,), jnp.float32)

        @pl.loop(0, ZROWS)
        def _(r):
            zbuf[r, pl.ds(0, 16)] = zero16
            zbuf[r, pl.ds(16, 16)] = zero16
            zbuf[r, pl.ds(24, 16)] = zero16

        @pl.loop(0, 5)
        def _(q):
            pltpu.sync_copy(zbuf, acc.at[pl.ds(s * SROWS + q * ZROWS, ZROWS)])

        plsc.subcore_barrier()

        def loads(t, b):
            r0 = s * rows_per_sub + t * KS2
            return (pltpu.make_async_copy(dst_hbm.at[pl.ds(r0, KS2)],
                                          idx_v.at[b], lsem),
                    pltpu.make_async_copy(
                        msg_hbm.at[pl.ds(r0 * 128, CHS), pl.ds(0, D_MSG)],
                        msg_v.at[b], lsem))

        for cp in loads(0, 0):
            cp.start()

        @pl.loop(0, nchunk, step=2)
        def _(t):
            for half, b in ((0, 0), (1, 1)):
                tt = t + half
                for cp in loads(tt, b):
                    cp.wait()

                @pl.when(tt + 1 < nchunk)
                def _():
                    for cp in loads(tt + 1, 1 - b):
                        cp.start()

                @pl.loop(0, KS2)
                def _(r):
                    @pl.loop(0, 8)
                    def _(kk):
                        v = idx_v[b, r, pl.ds(kk * 16, 16)] - lo
                        ok = (v >= 0) & (v < NHALF)
                        idx_v[b, r, pl.ds(kk * 16, 16)] = jnp.where(ok, v, NHALF)

                cps = [
                    pltpu.async_copy(msg_v.at[b, pl.ds(j * 128, 128)],
                                     acc.at[idx_v.at[b, j]], asem, add=True)
                    for j in range(KS2)
                ]
                for cp in cps:
                    cp.wait()

        plsc.subcore_barrier()

        @pl.loop(0, 5)
        def _(q):
            r = s * SROWS + q * ZROWS
            pltpu.sync_copy(acc.at[pl.ds(r, ZROWS)],
                            out_hbm.at[c, pl.ds(r, ZROWS)])

    return k(msg, dst2d)


def _node_out_kernel(tab_ref, pp_ref, pp2_ref, batch_ref,
                     ln0g_ref, ln0b_ref,
                     f1h_ref, f1w_ref, f1wb_ref, f1gw_ref, f1gb_ref, f1v_ref,
                     f2h_ref, f2w_ref, f2wb_ref, f2gw_ref, f2gb_ref, f2v_ref,
                     ln1g_ref, ln1b_ref, lnog_ref, lnob_ref,
                     oh_ref, ow_ref, owb_ref, o_ref, acc_ref, *, nsteps):
    i = pl.program_id(0)

    @pl.when(i == 0)
    def _():
        acc_ref[...] = jnp.zeros_like(acc_ref)

    psT = (pp_ref[0] + pp2_ref[0]).T
    hsT = tab_ref[...].T[0:8]
    cnt = jnp.maximum(psT[32:33], 1.0)
    inv = 1.0 / cnt
    agg_s = psT[0:8] * inv
    V = [psT[8:16] * inv, psT[16:24] * inv, psT[24:32] * inv]

    def gvp_ln(sv, Vv, g, b):
        mu = jnp.mean(sv, axis=0, keepdims=True)
        var = jnp.mean((sv - mu) ** 2, axis=0, keepdims=True)
        sn = (sv - mu) / jnp.sqrt(var + 1e-5) * g + b
        nsq = jnp.maximum(Vv[0] ** 2 + Vv[1] ** 2 + Vv[2] ** 2, 1e-8)
        den = 1.0 / jnp.sqrt(jnp.mean(nsq, axis=0, keepdims=True))
        return sn, [v * den for v in Vv]

    def gvp(sv, Vv, wh, ws, wsb, wsv, wsvb, wv, act):
        H = [_dotT(wh, v) for v in Vv]
        vn = jnp.sqrt(jnp.maximum(H[0] ** 2 + H[1] ** 2 + H[2] ** 2, 1e-8))
        so = _dotT(ws, jnp.concatenate([sv, vn], axis=0)) + wsb
        gate = jax.nn.sigmoid(_dotT(wsv, so) + wsvb)
        Vo = [_dotT(wv, h) * gate for h in H]
        if act is not None:
            so = act(so)
        return so, Vo

    xs, Xv = gvp_ln(hsT + agg_s, V, ln0g_ref[...], ln0b_ref[...])
    fs1, Fv1 = gvp(xs, Xv, f1h_ref[...], f1w_ref[...], f1wb_ref[...],
                   f1gw_ref[...], f1gb_ref[...], f1v_ref[...], _silu)
    fs2, Fv2 = gvp(fs1, Fv1, f2h_ref[...], f2w_ref[...], f2wb_ref[...],
                   f2gw_ref[...], f2gb_ref[...], f2v_ref[...], None)
    ys, Yv = gvp_ln(xs + fs2, [Xv[d] + Fv2[d] for d in range(3)],
                    ln1g_ref[...], ln1b_ref[...])
    os_, Ov = gvp_ln(ys, Yv, lnog_ref[...], lnob_ref[...])
    OH = [_dotT(oh_ref[...], v) for v in Ov]
    vn3 = jnp.sqrt(jnp.maximum(OH[0] ** 2 + OH[1] ** 2 + OH[2] ** 2, 1e-8))
    out_sT = _dotT(ow_ref[...], jnp.concatenate([os_, vn3], axis=0)) \
        + owb_ref[...]

    bb = batch_ref[0]
    onehot = (lax.broadcasted_iota(jnp.int32, (NG, bb.shape[1]), 0)
              == bb).astype(jnp.float32)
    sums = lax.dot_general(out_sT, onehot, (((1,), (1,)), ((), ())),
                           preferred_element_type=jnp.float32)
    cnts = lax.dot_general(jnp.ones((1, bb.shape[1]), jnp.float32), onehot,
                           (((1,), (1,)), ((), ())),
                           preferred_element_type=jnp.float32)
    acc_ref[0:2, 0:NG] += sums
    acc_ref[2:3, 0:NG] += cnts

    @pl.when(i == nsteps - 1)
    def _():
        o_ref[...] = (acc_ref[0:2, 0:NG]
                      / jnp.maximum(acc_ref[2:3, 0:NG], 1.0)).T


def _node_out(table, partials, partials2, batch3d, p):
    f1, f2, go = p['ff1'], p['ff2'], p['gvp_out']
    nsteps = N // BN5
    blocks_per_core = NHALF // BN5
    pspec = pl.BlockSpec((1, BN5, D_MSG),
                         lambda i: (i // blocks_per_core,
                                    i % blocks_per_core, 0))
    return pl.pallas_call(
        functools.partial(_node_out_kernel, nsteps=nsteps),
        grid=(nsteps,),
        in_specs=[
            pl.BlockSpec((BN5, D_TAB), lambda i: (i, 0)),
            pspec, pspec,
            pl.BlockSpec((1, 1, BN5), lambda i: (i, 0, 0)),
            _full((8, 1)), _full((8, 1)),
            _full((8, 16)), _full((24, 32)), _full((32, 1)), _full((32, 16)),
            _full((16, 1)), _full((16, 16)),
            _full((16, 16)), _full((48, 8)), _full((8, 1)), _full((8, 8)),
            _full((8, 1)), _full((16, 8)),
            _full((8, 1)), _full((8, 1)), _full((8, 1)), _full((8, 1)),
            _full((8, 8)), _full((16, 2)), _full((2, 1)),
        ],
        out_specs=pl.BlockSpec((NG, 2), lambda i: (0, 0)),
        out_shape=jax.ShapeDtypeStruct((NG, 2), jnp.float32),
        scratch_shapes=[pltpu.VMEM((8, 128), jnp.float32)],
    )(table, partials, partials2, batch3d,
      p['ln0_g'].reshape(8, 1), p['ln0_b'].reshape(8, 1),
      f1['wh'], f1['ws_w'], f1['ws_b'].reshape(32, 1), f1['wsv_w'],
      f1['wsv_b'].reshape(16, 1), f1['wv'],
      f2['wh'], f2['ws_w'], f2['ws_b'].reshape(8, 1), f2['wsv_w'],
      f2['wsv_b'].reshape(8, 1), f2['wv'],
      p['ln1_g'].reshape(8, 1), p['ln1_b'].reshape(8, 1),
      p['ln_out_g'].reshape(8, 1), p['ln_out_b'].reshape(8, 1),
      go['wh'], go['ws_w'], go['ws_b'].reshape(2, 1))


def kernel(positions, shifts, node_attrs, edge_index, batch, params):
    src = edge_index[0]
    dst = edge_index[1]
    pad = jnp.zeros((EP - E,), jnp.int32)
    src2d = jnp.concatenate([src, pad]).reshape(EP // 128, 128)
    dst2d = jnp.concatenate([dst, pad]).reshape(EP // 128, 128)

    table = _node_table(node_attrs, positions, params)

    def perm(a2d):
        return (a2d.reshape(-1, 8, 1024).transpose(0, 2, 1)
                .reshape(GROWS, 128))

    parts = []
    for h in range(2):
        s2 = src2d[h * GROWS:(h + 1) * GROWS]
        d2 = dst2d[h * GROWS:(h + 1) * GROWS]
        sf, df = _sc_gather(table, perm(s2), perm(d2))
        msg = _edge_msg(sf, df, params, h * EH)
        parts.append(_sc_scatter(msg, d2))
    return _node_out(table, parts[0], parts[1],
                     batch.reshape(N // BN5, 1, BN5), params)


# speedup vs baseline: 77.3651x; 1.0022x over previous
import functools

import jax
import jax.numpy as jnp
from jax import lax
from jax.experimental import pallas as pl
from jax.experimental.pallas import tpu as pltpu
from jax.experimental.pallas import tpu_sc as plsc

N = 50000
E = 800000
EP = 819200
NG = 32
NB = 8
CUT = 5.0

BN = 2000
BN5 = 1000
BE = 8192
D_TAB = 16
D_MSG = 40
CH = 1024
KSUB = 8
ROWS_W = 200


def _silu(x):
    return x * jax.nn.sigmoid(x)


def _ln(s, g, b):
    mu = jnp.mean(s, axis=-1, keepdims=True)
    var = jnp.mean((s - mu) ** 2, axis=-1, keepdims=True)
    return (s - mu) / jnp.sqrt(var + 1e-5) * g + b


def _full(shape):
    return pl.BlockSpec(shape, lambda i: tuple(0 for _ in shape))


def _node_table_kernel(na_ref, pos_ref, g_ref, b_ref, w_ref, wb_ref, o_ref):
    sn = _ln(na_ref[...], g_ref[...], b_ref[...])
    hs = jnp.dot(sn, w_ref[...], preferred_element_type=jnp.float32) + wb_ref[...]
    o_ref[...] = jnp.concatenate(
        [hs, pos_ref[...], jnp.zeros((hs.shape[0], 5), jnp.float32)], axis=1)


def _node_table(node_attrs, positions, p):
    gv = p['gvp_v']
    return pl.pallas_call(
        _node_table_kernel,
        grid=(N // BN,),
        in_specs=[
            pl.BlockSpec((BN, 4), lambda i: (i, 0)),
            pl.BlockSpec((BN, 3), lambda i: (i, 0)),
            _full((1, 4)), _full((1, 4)), _full((4, 8)), _full((1, 8)),
        ],
        out_specs=pl.BlockSpec((BN, D_TAB), lambda i: (i, 0)),
        out_shape=jax.ShapeDtypeStruct((N, D_TAB), jnp.float32),
    )(node_attrs, positions,
      p['ln_v_g'].reshape(1, 4), p['ln_v_b'].reshape(1, 4),
      gv['ws_w'], gv['ws_b'].reshape(1, 8))


EH = EP // 2
GROWS = EH // 128
GROWS_W = GROWS // 32
KG = 10
CHG = KG * 128


def _sc_gather(table, src2d, dst2d):
    mesh = plsc.VectorSubcoreMesh(core_axis_name="c", subcore_axis_name="s")

    @functools.partial(
        pl.kernel,
        mesh=mesh,
        out_type=(jax.ShapeDtypeStruct((EH, D_TAB), jnp.float32),
                  jax.ShapeDtypeStruct((EH, D_TAB), jnp.float32)),
        scratch_types=[
            pltpu.VMEM((2, KG, 128), jnp.int32),
            pltpu.VMEM((2, CHG, D_TAB), jnp.float32),
            pltpu.SemaphoreType.DMA,
            pltpu.SemaphoreType.DMA,
            pltpu.SemaphoreType.DMA,
        ],
        compiler_params=pltpu.CompilerParams(use_tc_tiling_on_sc=False),
    )
    def k(tab_hbm, src_hbm, dst_hbm, osrc_hbm, odst_hbm, idx_v, rows_v,
          isem, gsem, wsem):
        w = lax.axis_index("c") * 16 + lax.axis_index("s")

        for i_hbm, o_hbm in ((src_hbm, osrc_hbm), (dst_hbm, odst_hbm)):
            def idx_cp(t, b):
                return pltpu.make_async_copy(
                    i_hbm.at[pl.ds(w * GROWS_W + t * KG, KG)],
                    idx_v.at[b], isem)

            def wb_cp(t, b):
                return pltpu.make_async_copy(
                    rows_v.at[b],
                    o_hbm.at[pl.ds((w * GROWS_W + t * KG) * 128, CHG)], wsem)

            idx_cp(0, 0).start()

            @pl.loop(0, GROWS_W // KG, step=2)
            def _(t):
                for half, b in ((0, 0), (1, 1)):
                    tt = t + half

                    @pl.when(tt > 1)
                    def _():
                        wb_cp(tt, b).wait()

                    idx_cp(tt, b).wait()

                    @pl.when(tt + 1 < GROWS_W // KG)
                    def _():
                        idx_cp(tt + 1, 1 - b).start()

                    cps = [
                        pltpu.async_copy(tab_hbm.at[idx_v.at[b, j]],
                                         rows_v.at[b, pl.ds(j * 128, 128)],
                                         gsem)
                        for j in range(KG)
                    ]
                    for cp in cps:
                        cp.wait()
                    wb_cp(tt, b).start()

            wb_cp(0, 0).wait()
            wb_cp(1, 1).wait()

    return k(table, src2d, dst2d)


def _dotT(w, x):
    return lax.dot_general(w, x, (((0,), (0,)), ((), ())),
                           preferred_element_type=jnp.float32)


def _edge_msg_kernel(sf_ref, df_ref,
                     lneg_ref, lneb_ref, gew_ref, gewb_ref, gegw_ref,
                     scal_ref, wh8sq_ref, m1w_ref, m1wb_ref, m1gw_ref,
                     m1gb_ref, c1_ref, m2h_ref, m2w_ref, m2wb_ref,
                     m2gw_ref, m2gb_ref, m2v_ref, o_ref, *, base):
    i = pl.program_id(0)

    def unpack(ref):
        xt = ref[...].T
        return jnp.concatenate(
            [xt[16 * j:16 * j + 16, :] for j in range(8)], axis=1)

    sfT = unpack(sf_ref)[0:11]
    dfT = unpack(df_ref)[0:11]
    a_e = scal_ref[0, 0]
    b_e = scal_ref[0, 1]
    ge_gate_b = scal_ref[0, 2]

    hss, pos_s = sfT[0:8], sfT[8:11]
    hsd, pos_d = dfT[0:8], dfT[8:11]
    vec = pos_d - pos_s
    lsq = jnp.maximum(jnp.sum(vec * vec, axis=0, keepdims=True), 1e-12)
    length = jnp.sqrt(lsq)
    inv_len = 1.0 / length
    unit = vec * inv_len
    wfreq = ((lax.broadcasted_iota(jnp.int32, (NB, 1), 0) + 1).astype(jnp.float32)
             * (jnp.pi / CUT))
    bess = jnp.sqrt(2.0 / CUT) * jnp.sin(length * wfreq) * inv_len
    u = length * (1.0 / CUT)
    u3 = u * u * u
    u6 = u3 * u3
    u7 = u6 * u
    u8 = u7 * u
    env = (1.0 - 28.0 * u6 + 48.0 * u7 - 21.0 * u8) * (length < CUT).astype(jnp.float32)
    edge_s = bess * env

    mu = jnp.mean(edge_s, axis=0, keepdims=True)
    var = jnp.mean((edge_s - mu) ** 2, axis=0, keepdims=True)
    es0 = (edge_s - mu) / jnp.sqrt(var + 1e-5) * lneg_ref[...] + lneb_ref[...]
    nsq = jnp.sum(unit * unit, axis=0, keepdims=True)
    ev0 = unit / jnp.sqrt(jnp.maximum(nsq, 1e-8))
    ev0sq = jnp.sum(ev0 * ev0, axis=0, keepdims=True)
    vn_e = jnp.sqrt(jnp.maximum(ev0sq * a_e, 1e-8))
    es = _dotT(gew_ref[...], jnp.concatenate([es0, vn_e], axis=0)) \
        + gewb_ref[...]
    gate_e = jax.nn.sigmoid(
        jnp.sum(es * gegw_ref[...], axis=0, keepdims=True) + ge_gate_b)
    evv = ev0 * (b_e * gate_e)
    evvsq = jnp.sum(evv * evv, axis=0, keepdims=True)

    vn1 = jnp.sqrt(jnp.maximum(evvsq * wh8sq_ref[...], 1e-8))
    x41 = jnp.concatenate([hss, es, hsd, vn1], axis=0)
    s1 = _dotT(m1w_ref[...], x41) + m1wb_ref[...]
    gate1 = jax.nn.sigmoid(_dotT(m1gw_ref[...], s1) + m1gb_ref[...])
    g1 = c1_ref[...] * gate1
    s1 = _silu(s1)

    h2 = _dotT(m2h_ref[...], g1)
    vn2 = jnp.sqrt(jnp.maximum(evvsq * h2 * h2, 1e-8))
    ms = _dotT(m2w_ref[...], jnp.concatenate([s1, vn2], axis=0)) \
        + m2wb_ref[...]
    gate2 = jax.nn.sigmoid(_dotT(m2gw_ref[...], ms) + m2gb_ref[...])
    g2 = _dotT(m2v_ref[...], h2) * gate2

    gid = lax.broadcasted_iota(jnp.int32, (1, BE), 1) + i * BE + base
    valid = (gid < E).astype(jnp.float32)
    outT = jnp.concatenate(
        [ms, g2 * evv[0:1], g2 * evv[1:2], g2 * evv[2:3],
         valid, jnp.zeros((7, BE), jnp.float32)], axis=0) * valid
    o_ref[:, 0:D_MSG] = outT.T


def _edge_msg(srcf, dstf, p, base):
    ge, m1, m2 = p['gvp_e'], p['msg1'], p['msg2']
    wh8 = m1['wh'][8, :]
    scal = jnp.stack([ge['wh'][0, 0] ** 2,
                      ge['wh'][0, 0] * ge['wv'][0, 0],
                      ge['wsv_b'][0]]).reshape(1, 3)
    return pl.pallas_call(
        functools.partial(_edge_msg_kernel, base=base),
        grid=(EH // BE,),
        in_specs=[
            pl.BlockSpec((BE * D_TAB // 128, 128), lambda i: (i, 0)),
            pl.BlockSpec((BE * D_TAB // 128, 128), lambda i: (i, 0)),
            _full((8, 1)), _full((8, 1)), _full((9, 8)), _full((8, 1)),
            _full((8, 1)), _full((1, 3)), _full((17, 1)), _full((41, 8)),
            _full((8, 1)), _full((8, 8)), _full((8, 1)), _full((8, 1)),
            _full((8, 8)), _full((16, 8)), _full((8, 1)), _full((8, 8)),
            _full((8, 1)), _full((8, 8)),
        ],
        out_specs=pl.BlockSpec((BE, 128), lambda i: (i, 0)),
        out_shape=jax.ShapeDtypeStruct((EH, 128), jnp.float32),
    )(srcf.reshape(EH * D_TAB // 128, 128),
      dstf.reshape(EH * D_TAB // 128, 128),
      p['ln_e_g'].reshape(8, 1), p['ln_e_b'].reshape(8, 1),
      ge['ws_w'], ge['ws_b'].reshape(8, 1), ge['wsv_w'].reshape(8, 1),
      scal, (wh8 * wh8).reshape(17, 1),
      m1['ws_w'], m1['ws_b'].reshape(8, 1), m1['wsv_w'],
      m1['wsv_b'].reshape(8, 1), (wh8 @ m1['wv']).reshape(8, 1),
      m2['wh'], m2['ws_w'], m2['ws_b'].reshape(8, 1), m2['wsv_w'],
      m2['wsv_b'].reshape(8, 1), m2['wv'])


NHALF = N // 2
NACC = 26000
ZROWS = 325
SROWS = NACC // 16


def _sc_scatter(msg, dst2d):
    mesh = plsc.VectorSubcoreMesh(core_axis_name="c", subcore_axis_name="s")
    rows_per_sub = GROWS // 16

    KS2 = 4
    CHS = KS2 * 128
    nchunk = rows_per_sub // KS2

    @functools.partial(
        pl.kernel,
        mesh=mesh,
        out_type=jax.ShapeDtypeStruct((2, NACC, D_MSG), jnp.float32),
        scratch_types=[
            pltpu.VMEM((2, KS2, 128), jnp.int32),
            pltpu.VMEM((2, CHS, D_MSG), jnp.float32),
            pltpu.VMEM((ZROWS, D_MSG), jnp.float32),
            pltpu.VMEM_SHARED((NACC, D_MSG), jnp.float32),
            pltpu.SemaphoreType.DMA,
            pltpu.SemaphoreType.DMA,
        ],
        compiler_params=pltpu.CompilerParams(use_tc_tiling_on_sc=False),
    )
    def k(msg_hbm, dst_hbm, out_hbm, idx_v, msg_v, zbuf, acc, lsem, asem):
        c = lax.axis_index("c")
        s = lax.axis_index("s")
        lo = c * NHALF
        zero16 = jnp.zeros((16,), jnp.float32)

        @pl.loop(0, ZROWS)
        def _(r):
            zbuf[r, pl.ds(0, 16)] = zero16
            zbuf[r, pl.ds(16, 16)] = zero16
            zbuf[r, pl.ds(24, 16)] = zero16

        @pl.loop(0, 5)
        def _(q):
            pltpu.sync_copy(zbuf, acc.at[pl.ds(s * SROWS + q * ZROWS, ZROWS)])

        plsc.subcore_barrier()

        def loads(t, b):
            r0 = s * rows_per_sub + t * KS2
            return (pltpu.make_async_copy(dst_hbm.at[pl.ds(r0, KS2)],
                                          idx_v.at[b], lsem),
                    pltpu.make_async_copy(
                        msg_hbm.at[pl.ds(r0 * 128, CHS), pl.ds(0, D_MSG)],
                        msg_v.at[b], lsem))

        for cp in loads(0, 0):
            cp.start()

        @pl.loop(0, nchunk, step=2)
        def _(t):
            for half, b in ((0, 0), (1, 1)):
                tt = t + half
                for cp in loads(tt, b):
                    cp.wait()

                @pl.when(tt + 1 < nchunk)
                def _():
                    for cp in loads(tt + 1, 1 - b):
                        cp.start()

                @pl.loop(0, KS2)
                def _(r):
                    @pl.loop(0, 8)
                    def _(kk):
                        v = idx_v[b, r, pl.ds(kk * 16, 16)] - lo
                        ok = (v >= 0) & (v < NHALF)
                        idx_v[b, r, pl.ds(kk * 16, 16)] = jnp.where(ok, v, NHALF)

                cps = [
                    pltpu.async_copy(msg_v.at[b, pl.ds(j * 128, 128)],
                                     acc.at[idx_v.at[b, j]], asem, add=True)
                    for j in range(KS2)
                ]
                for cp in cps:
                    cp.wait()

        plsc.subcore_barrier()

        @pl.loop(0, 5)
        def _(q):
            r = s * SROWS + q * ZROWS
            pltpu.sync_copy(acc.at[pl.ds(r, ZROWS)],
                            out_hbm.at[c, pl.ds(r, ZROWS)])

    return k(msg, dst2d)


def _node_out_kernel(tab_ref, pp_ref, pp2_ref, batch_ref,
                     ln0g_ref, ln0b_ref,
                     f1h_ref, f1w_ref, f1wb_ref, f1gw_ref, f1gb_ref, f1v_ref,
                     f2h_ref, f2w_ref, f2wb_ref, f2gw_ref, f2gb_ref, f2v_ref,
                     ln1g_ref, ln1b_ref, lnog_ref, lnob_ref,
                     oh_ref, ow_ref, owb_ref, o_ref, acc_ref, *, nsteps):
    i = pl.program_id(0)

    @pl.when(i == 0)
    def _():
        acc_ref[...] = jnp.zeros_like(acc_ref)

    psT = (pp_ref[0] + pp2_ref[0]).T
    hsT = tab_ref[...].T[0:8]
    cnt = jnp.maximum(psT[32:33], 1.0)
    inv = 1.0 / cnt
    agg_s = psT[0:8] * inv
    V = [psT[8:16] * inv, psT[16:24] * inv, psT[24:32] * inv]

    def gvp_ln(sv, Vv, g, b):
        mu = jnp.mean(sv, axis=0, keepdims=True)
        var = jnp.mean((sv - mu) ** 2, axis=0, keepdims=True)
        sn = (sv - mu) / jnp.sqrt(var + 1e-5) * g + b
        nsq = jnp.maximum(Vv[0] ** 2 + Vv[1] ** 2 + Vv[2] ** 2, 1e-8)
        den = 1.0 / jnp.sqrt(jnp.mean(nsq, axis=0, keepdims=True))
        return sn, [v * den for v in Vv]

    def gvp(sv, Vv, wh, ws, wsb, wsv, wsvb, wv, act):
        H = [_dotT(wh, v) for v in Vv]
        vn = jnp.sqrt(jnp.maximum(H[0] ** 2 + H[1] ** 2 + H[2] ** 2, 1e-8))
        so = _dotT(ws, jnp.concatenate([sv, vn], axis=0)) + wsb
        gate = jax.nn.sigmoid(_dotT(wsv, so) + wsvb)
        Vo = [_dotT(wv, h) * gate for h in H]
        if act is not None:
            so = act(so)
        return so, Vo

    xs, Xv = gvp_ln(hsT + agg_s, V, ln0g_ref[...], ln0b_ref[...])
    fs1, Fv1 = gvp(xs, Xv, f1h_ref[...], f1w_ref[...], f1wb_ref[...],
                   f1gw_ref[...], f1gb_ref[...], f1v_ref[...], _silu)
    fs2, Fv2 = gvp(fs1, Fv1, f2h_ref[...], f2w_ref[...], f2wb_ref[...],
                   f2gw_ref[...], f2gb_ref[...], f2v_ref[...], None)
    ys, Yv = gvp_ln(xs + fs2, [Xv[d] + Fv2[d] for d in range(3)],
                    ln1g_ref[...], ln1b_ref[...])
    os_, Ov = gvp_ln(ys, Yv, lnog_ref[...], lnob_ref[...])
    OH = [_dotT(oh_ref[...], v) for v in Ov]
    vn3 = jnp.sqrt(jnp.maximum(OH[0] ** 2 + OH[1] ** 2 + OH[2] ** 2, 1e-8))
    out_sT = _dotT(ow_ref[...], jnp.concatenate([os_, vn3], axis=0)) \
        + owb_ref[...]

    bb = batch_ref[0]
    onehot = (lax.broadcasted_iota(jnp.int32, (NG, bb.shape[1]), 0)
              == bb).astype(jnp.float32)
    sums = lax.dot_general(out_sT, onehot, (((1,), (1,)), ((), ())),
                           preferred_element_type=jnp.float32)
    cnts = lax.dot_general(jnp.ones((1, bb.shape[1]), jnp.float32), onehot,
                           (((1,), (1,)), ((), ())),
                           preferred_element_type=jnp.float32)
    acc_ref[0:2, 0:NG] += sums
    acc_ref[2:3, 0:NG] += cnts

    @pl.when(i == nsteps - 1)
    def _():
        o_ref[...] = (acc_ref[0:2, 0:NG]
                      / jnp.maximum(acc_ref[2:3, 0:NG], 1.0)).T


def _node_out(table, partials, partials2, batch3d, p):
    f1, f2, go = p['ff1'], p['ff2'], p['gvp_out']
    nsteps = N // BN5
    blocks_per_core = NHALF // BN5
    pspec = pl.BlockSpec((1, BN5, D_MSG),
                         lambda i: (i // blocks_per_core,
                                    i % blocks_per_core, 0))
    return pl.pallas_call(
        functools.partial(_node_out_kernel, nsteps=nsteps),
        grid=(nsteps,),
        in_specs=[
            pl.BlockSpec((BN5, D_TAB), lambda i: (i, 0)),
            pspec, pspec,
            pl.BlockSpec((1, 1, BN5), lambda i: (i, 0, 0)),
            _full((8, 1)), _full((8, 1)),
            _full((8, 16)), _full((24, 32)), _full((32, 1)), _full((32, 16)),
            _full((16, 1)), _full((16, 16)),
            _full((16, 16)), _full((48, 8)), _full((8, 1)), _full((8, 8)),
            _full((8, 1)), _full((16, 8)),
            _full((8, 1)), _full((8, 1)), _full((8, 1)), _full((8, 1)),
            _full((8, 8)), _full((16, 2)), _full((2, 1)),
        ],
        out_specs=pl.BlockSpec((NG, 2), lambda i: (0, 0)),
        out_shape=jax.ShapeDtypeStruct((NG, 2), jnp.float32),
        scratch_shapes=[pltpu.VMEM((8, 128), jnp.float32)],
    )(table, partials, partials2, batch3d,
      p['ln0_g'].reshape(8, 1), p['ln0_b'].reshape(8, 1),
      f1['wh'], f1['ws_w'], f1['ws_b'].reshape(32, 1), f1['wsv_w'],
      f1['wsv_b'].reshape(16, 1), f1['wv'],
      f2['wh'], f2['ws_w'], f2['ws_b'].reshape(8, 1), f2['wsv_w'],
      f2['wsv_b'].reshape(8, 1), f2['wv'],
      p['ln1_g'].reshape(8, 1), p['ln1_b'].reshape(8, 1),
      p['ln_out_g'].reshape(8, 1), p['ln_out_b'].reshape(8, 1),
      go['wh'], go['ws_w'], go['ws_b'].reshape(2, 1))


def kernel(positions, shifts, node_attrs, edge_index, batch, params):
    src = edge_index[0]
    dst = edge_index[1]
    pad = jnp.zeros((EP - E,), jnp.int32)
    src2d = jnp.concatenate([src, pad]).reshape(EP // 128, 128)
    dst2d = jnp.concatenate([dst, pad]).reshape(EP // 128, 128)

    table = _node_table(node_attrs, positions, params)

    def perm(a2d):
        return (a2d.reshape(-1, 8, 1024).transpose(0, 2, 1)
                .reshape(GROWS, 128))

    parts = []
    for h in range(2):
        s2 = src2d[h * GROWS:(h + 1) * GROWS]
        d2 = dst2d[h * GROWS:(h + 1) * GROWS]
        sf, df = _sc_gather(table, perm(s2), perm(d2))
        msg = _edge_msg(sf, df, params, h * EH)
        parts.append(_sc_scatter(msg, d2))
    return _node_out(table, parts[0], parts[1],
                     batch.reshape(N // BN5, 1, BN5), params)
